# baseline (speedup 1.0000x reference)
import functools

import jax
import jax.numpy as jnp
from jax import lax
from jax.experimental import pallas as pl
from jax.experimental.pallas import tpu as pltpu
from jax.experimental.pallas import tpu_sc as plsc

N, E, B, H, D = 50000, 800000, 64, 64, 128
NC, NS, L = 2, 16, 16
NW = NC * NS
C = 64
C_GIN = 512
C_SCA = 512
C_GCV = 256
NP = 50176
EP = 802816
NCHUNKS = EP // C
CH_W = NCHUNKS // NW
CH_S = NCHUNKS // NS
RPS = NP // NS
PN_W = NP // NW
PCH = 392
BP = B + 1

_mesh = plsc.VectorSubcoreMesh(core_axis_name="c", subcore_axis_name="s")


def _iota16():
    return lax.iota(jnp.int32, 16)


def _fill2d(ref, rows, cols, value):
    it = _iota16()
    v = jnp.full((16,), value, jnp.float32)
    ngroups = rows * cols // 16

    @pl.loop(0, ngroups)
    def _(g):
        flat = g * 16 + it
        plsc.store_scatter(ref, [flat // cols, flat % cols], v)


ZR = 16


def _zero_spmem(zbuf, accum, s, rows, cols):
    _fill2d(zbuf, ZR, cols, 0.0)
    nblk = rows // ZR

    @pl.loop(0, nblk)
    def _(t):
        pltpu.sync_copy(zbuf, accum.at[pl.ds(s * rows + t * ZR, ZR)])


def _make_seg_sum(F, split_by_worker, CK):
    nch = EP // CK // (NW if split_by_worker else NS)

    def body(table, srcr, dstr, out, idx_s, idx_d, rows, zbuf, accum,
             sem0, sem1):
        c = lax.axis_index("c")
        s = lax.axis_index("s")
        _zero_spmem(zbuf, accum, s, RPS, F)
        plsc.subcore_barrier()
        if split_by_worker:
            base = (c * NS + s) * nch
        else:
            base = s * nch
        sems = (sem0, sem1)

        def prep(slot, j, sem):
            pltpu.sync_copy(srcr.at[base + j], idx_s.at[slot])
            pltpu.sync_copy(dstr.at[base + j], idx_d.at[slot])
            return pltpu.async_copy(table.at[idx_s.at[slot]], rows.at[slot],
                                    sem)

        def drain(slot, sem):
            pltpu.make_async_copy(table.at[idx_s.at[slot]], rows.at[slot],
                                  sem).wait()
            pltpu.sync_copy(rows.at[slot], accum.at[idx_d.at[slot]], add=True)

        prep(0, 0, sem0)

        @pl.loop(0, nch // 2)
        def _(t):
            prep(1, 2 * t + 1, sem1)
            drain(0, sem0)

            @pl.when(t < nch // 2 - 1)
            def _():
                prep(0, 2 * t + 2, sem0)

            drain(1, sem1)

        plsc.subcore_barrier()
        pltpu.sync_copy(accum.at[pl.ds(s * RPS, RPS)],
                        out.at[c, pl.ds(s * RPS, RPS)])

    return pl.kernel(
        body,
        out_type=jax.ShapeDtypeStruct((NC, NP, F), jnp.float32),
        mesh=_mesh,
        compiler_params=pltpu.CompilerParams(use_tc_tiling_on_sc=False, needs_layout_passes=False),
        scratch_types=[
            pltpu.VMEM((2, CK), jnp.int32),
            pltpu.VMEM((2, CK), jnp.int32),
            pltpu.VMEM((2, CK, F), jnp.float32),
            pltpu.VMEM((ZR, F), jnp.float32),
            pltpu.VMEM_SHARED((NP, F), jnp.float32),
            pltpu.SemaphoreType.DMA,
            pltpu.SemaphoreType.DMA,
        ],
    )


def _gat_edge_body(a_src, a_dst, srcr, dstr, ee_out, den_out,
                   idx_s, idx_d, abuf_s, abuf_d, eebuf, zbuf, accum,
                   sem0, sem1):
    c = lax.axis_index("c")
    s = lax.axis_index("s")
    _zero_spmem(zbuf, accum, s, RPS, 2)
    plsc.subcore_barrier()
    base = (c * NS + s) * (EP // C_SCA // NW)
    it = _iota16()
    row_h = it // 2
    col_h = it % 2

    def prep(slot, j, sem):
        pltpu.sync_copy(srcr.at[base + j], idx_s.at[slot])
        pltpu.sync_copy(dstr.at[base + j], idx_d.at[slot])
        pltpu.async_copy(a_src.at[idx_s.at[slot]], abuf_s.at[slot], sem)
        pltpu.async_copy(a_dst.at[idx_d.at[slot]], abuf_d.at[slot], sem)

    def drain(slot, j, sem):
        pltpu.make_async_copy(a_src.at[idx_s.at[slot]], abuf_s.at[slot],
                              sem).wait()
        pltpu.make_async_copy(a_dst.at[idx_d.at[slot]], abuf_d.at[slot],
                              sem).wait()

        @pl.loop(0, 2 * C_SCA // 16, unroll=4)
        def _(g):
            r = row_h + g * 8
            vs = plsc.load_gather(abuf_s.at[slot], [r, col_h])
            vd = plsc.load_gather(abuf_d.at[slot], [r, col_h])
            e = vs + vd
            e = jnp.maximum(e, 0.2 * e)
            plsc.store_scatter(eebuf.at[slot], [r, col_h], jnp.exp(e))

        pltpu.sync_copy(eebuf.at[slot], accum.at[idx_d.at[slot]], add=True)
        pltpu.sync_copy(eebuf.at[slot],
                        ee_out.at[pl.ds((base + j) * C_SCA, C_SCA)])

    nch_a = EP // C_SCA // NW
    prep(0, 0, sem0)

    @pl.loop(0, nch_a // 2)
    def _(t):
        prep(1, 2 * t + 1, sem1)
        drain(0, 2 * t, sem0)

        @pl.when(t < nch_a // 2 - 1)
        def _():
            prep(0, 2 * t + 2, sem0)

        drain(1, 2 * t + 1, sem1)

    plsc.subcore_barrier()
    pltpu.sync_copy(accum.at[pl.ds(s * RPS, RPS)],
                    den_out.at[c, pl.ds(s * RPS, RPS)])


_gat_edge = pl.kernel(
    _gat_edge_body,
    out_type=(jax.ShapeDtypeStruct((EP, 2), jnp.float32),
              jax.ShapeDtypeStruct((NC, NP, 2), jnp.float32)),
    mesh=_mesh,
    compiler_params=pltpu.CompilerParams(use_tc_tiling_on_sc=False, needs_layout_passes=False),
    scratch_types=[
        pltpu.VMEM((2, C_SCA), jnp.int32),
        pltpu.VMEM((2, C_SCA), jnp.int32),
        pltpu.VMEM((2, C_SCA, 2), jnp.float32),
        pltpu.VMEM((2, C_SCA, 2), jnp.float32),
        pltpu.VMEM((2, C_SCA, 2), jnp.float32),
        pltpu.VMEM((ZR, 2), jnp.float32),
        pltpu.VMEM_SHARED((NP, 2), jnp.float32),
        pltpu.SemaphoreType.DMA,
        pltpu.SemaphoreType.DMA,
    ],
)


def _gat_msg_body(y0, y1, ee, den, srcr, dstr, out,
                  idx_s, idx_d, idx_sc, rows, eebuf, denbuf, wbuf, msg, zbuf,
                  accum, sem0, sem1, sem2, sem3):
    c = lax.axis_index("c")
    s = lax.axis_index("s")
    _zero_spmem(zbuf, accum, s, RPS, 32)
    plsc.subcore_barrier()
    base = s * CH_S
    it = _iota16()
    row_h = it // 2
    col_h = it % 2
    gsems = (sem0, sem1)
    ssems = (sem2, sem3)

    def prep(slot, j):
        pltpu.sync_copy(srcr.at[base + j], idx_s.at[slot])
        pltpu.sync_copy(dstr.at[base + j], idx_d.at[slot])

        @pl.when(c == 0)
        def _():
            pltpu.async_copy(y0.at[idx_s.at[slot]], rows.at[slot],
                             gsems[slot])

        @pl.when(c == 1)
        def _():
            pltpu.async_copy(y1.at[idx_s.at[slot]], rows.at[slot],
                             gsems[slot])

        pltpu.async_copy(den.at[idx_d.at[slot]], denbuf.at[slot],
                         gsems[slot])
        pltpu.sync_copy(ee.at[pl.ds((base + j) * C, C)], eebuf.at[slot])

    def drain(slot, t):
        pltpu.make_async_copy(y0.at[idx_s.at[slot]], rows.at[slot],
                              gsems[slot]).wait()
        pltpu.make_async_copy(den.at[idx_d.at[slot]], denbuf.at[slot],
                              gsems[slot]).wait()

        @pl.when(t > 0)
        def _():
            pltpu.make_async_copy(msg.at[slot], accum.at[idx_sc.at[slot]],
                                  ssems[slot]).wait()

        @pl.loop(0, 2 * C // 16, unroll=4)
        def _(g):
            r = row_h + g * 8
            ve = plsc.load_gather(eebuf.at[slot], [r, col_h])
            vd = plsc.load_gather(denbuf.at[slot], [r, col_h])
            plsc.store_scatter(wbuf, [r, col_h], ve / vd)

        @pl.loop(0, C, unroll=8)
        def _(k):
            vk = jnp.full((16,), k, jnp.int32)
            z = jnp.zeros((16,), jnp.int32)
            e0 = plsc.load_gather(wbuf, [vk, z])
            e1 = plsc.load_gather(wbuf, [vk, z + 1])
            r0 = rows[slot, k, pl.ds(0, 16)]
            r1 = rows[slot, k, pl.ds(16, 16)]
            r2 = rows[slot, k, pl.ds(32, 16)]
            r3 = rows[slot, k, pl.ds(48, 16)]
            msg[slot, k, pl.ds(0, 16)] = r0 * e0 + r2 * e1
            msg[slot, k, pl.ds(16, 16)] = r1 * e0 + r3 * e1

        @pl.loop(0, C // 16, unroll=8)
        def _(g):
            idx_sc[slot, pl.ds(g * 16, 16)] = idx_d[slot, pl.ds(g * 16, 16)]

        pltpu.async_copy(msg.at[slot], accum.at[idx_sc.at[slot]],
                         ssems[slot], add=True)

    prep(0, 0)

    @pl.loop(0, CH_S // 2)
    def _(t):
        prep(1, 2 * t + 1)
        drain(0, t)

        @pl.when(t < CH_S // 2 - 1)
        def _():
            prep(0, 2 * t + 2)

        drain(1, t)

    pltpu.make_async_copy(msg.at[0], accum.at[idx_sc.at[0]], sem2).wait()
    pltpu.make_async_copy(msg.at[1], accum.at[idx_sc.at[1]], sem3).wait()
    plsc.subcore_barrier()
    pltpu.sync_copy(accum.at[pl.ds(s * RPS, RPS)],
                    out.at[c, pl.ds(s * RPS, RPS)])


_gat_msg = pl.kernel(
    _gat_msg_body,
    out_type=jax.ShapeDtypeStruct((NC, NP, 32), jnp.float32),
    mesh=_mesh,
    compiler_params=pltpu.CompilerParams(use_tc_tiling_on_sc=False, needs_layout_passes=False),
    scratch_types=[
        pltpu.VMEM((2, C), jnp.int32),
        pltpu.VMEM((2, C), jnp.int32),
        pltpu.VMEM((2, C), jnp.int32),
        pltpu.VMEM((2, C, 64), jnp.float32),
        pltpu.VMEM((2, C, 2), jnp.float32),
        pltpu.VMEM((2, C, 2), jnp.float32),
        pltpu.VMEM((C, 2), jnp.float32),
        pltpu.VMEM((2, C, 32), jnp.float32),
        pltpu.VMEM((ZR, 32), jnp.float32),
        pltpu.VMEM_SHARED((NP, 32), jnp.float32),
        pltpu.SemaphoreType.DMA,
        pltpu.SemaphoreType.DMA,
        pltpu.SemaphoreType.DMA,
        pltpu.SemaphoreType.DMA,
    ],
)


def _gconv_body(h2a, h2b, srcr, dstr, out,
                idx_s, idx_d, rows, zbuf, accum, sem0, sem1):
    c = lax.axis_index("c")
    s = lax.axis_index("s")
    _zero_spmem(zbuf, accum, s, RPS, 32)
    plsc.subcore_barrier()
    nch_g = EP // C_GCV // NS
    base = s * nch_g

    def prep(slot, j, sem):
        pltpu.sync_copy(srcr.at[base + j], idx_s.at[slot])
        pltpu.sync_copy(dstr.at[base + j], idx_d.at[slot])

        @pl.when(c == 0)
        def _():
            pltpu.async_copy(h2a.at[idx_s.at[slot]], rows.at[slot], sem)

        @pl.when(c == 1)
        def _():
            pltpu.async_copy(h2b.at[idx_s.at[slot]], rows.at[slot], sem)

    def drain(slot, sem):
        pltpu.make_async_copy(h2a.at[idx_s.at[slot]], rows.at[slot],
                              sem).wait()
        pltpu.sync_copy(rows.at[slot], accum.at[idx_d.at[slot]], add=True)

    prep(0, 0, sem0)

    @pl.loop(0, nch_g // 2)
    def _(t):
        prep(1, 2 * t + 1, sem1)
        drain(0, sem0)

        @pl.when(t < nch_g // 2 - 1)
        def _():
            prep(0, 2 * t + 2, sem0)

        drain(1, sem1)

    plsc.subcore_barrier()
    pltpu.sync_copy(accum.at[pl.ds(s * RPS, RPS)],
                    out.at[c, pl.ds(s * RPS, RPS)])


_gconv = pl.kernel(
    _gconv_body,
    out_type=jax.ShapeDtypeStruct((NC, NP, 32), jnp.float32),
    mesh=_mesh,
    compiler_params=pltpu.CompilerParams(use_tc_tiling_on_sc=False, needs_layout_passes=False),
    scratch_types=[
        pltpu.VMEM((2, C_GCV), jnp.int32),
        pltpu.VMEM((2, C_GCV), jnp.int32),
        pltpu.VMEM((2, C_GCV, 32), jnp.float32),
        pltpu.VMEM((ZR, 32), jnp.float32),
        pltpu.VMEM_SHARED((NP, 32), jnp.float32),
        pltpu.SemaphoreType.DMA,
        pltpu.SemaphoreType.DMA,
    ],
)


def _pool_body(h3, bidx, sum_out, max_out, cnt_out,
               h3buf, bbuf, sacc, macc, cacc):
    c = lax.axis_index("c")
    s = lax.axis_index("s")
    w = c * NS + s
    _fill2d(sacc, BP, 64, 0.0)
    _fill2d(macc, BP, 64, float("-inf"))
    _fill2d(cacc, BP, 16, 0.0)

    it = _iota16()
    lane0 = it == 0
    ones = jnp.ones((16,), jnp.float32)

    @pl.loop(0, PN_W // PCH)
    def _(t):
        row0 = w * PN_W + t * PCH
        pltpu.sync_copy(h3.at[pl.ds(row0, PCH)], h3buf)
        pltpu.sync_copy(bidx.at[pl.ds(row0, PCH)], bbuf)

        @pl.loop(0, PCH)
        def _(i):
            vi = jnp.full((16,), i, jnp.int32)
            vb = plsc.load_gather(bbuf, [vi])
            plsc.addupdate_scatter(cacc, [vb, it], ones, mask=lane0)
            for k in range(4):
                col = k * 16 + it
                v = plsc.load_gather(h3buf, [vi, col])
                plsc.addupdate_scatter(sacc, [vb, col], v)
                cur = plsc.load_gather(macc, [vb, col])
                plsc.store_scatter(macc, [vb, col], jnp.maximum(cur, v))

    pltpu.sync_copy(sacc, sum_out.at[w])
    pltpu.sync_copy(macc, max_out.at[w])
    pltpu.sync_copy(cacc, cnt_out.at[w])


_pool = pl.kernel(
    _pool_body,
    out_type=(jax.ShapeDtypeStruct((NW, BP, 64), jnp.float32),
              jax.ShapeDtypeStruct((NW, BP, 64), jnp.float32),
              jax.ShapeDtypeStruct((NW, BP, 16), jnp.float32)),
    mesh=_mesh,
    compiler_params=pltpu.CompilerParams(use_tc_tiling_on_sc=False, needs_layout_passes=False),
    scratch_types=[
        pltpu.VMEM((PCH, 64), jnp.float32),
        pltpu.VMEM((PCH,), jnp.int32),
        pltpu.VMEM((BP, 64), jnp.float32),
        pltpu.VMEM((BP, 64), jnp.float32),
        pltpu.VMEM((BP, 16), jnp.float32),
    ],
)


def _gelu(v):
    return 0.5 * v * (1.0 + lax.erf(v * 0.7071067811865476))


BR = 1568
_GRID = NP // BR


def _tc1_body(x16, aggp, wg, bg, wgat, asrc, adst,
              y0_o, y1_o, as_o, ad_o, es_o):
    agg = aggp[0] + aggp[1]
    h = _gelu((x16[...] + agg) @ wg[...] + bg[...])
    wx = h @ wgat[...]
    wx0 = wx[:, :64]
    wx1 = wx[:, 64:]
    as0 = jnp.sum(wx0 * asrc[0, :][None, :], axis=1, keepdims=True)
    as1 = jnp.sum(wx1 * asrc[1, :][None, :], axis=1, keepdims=True)
    ad0 = jnp.sum(wx0 * adst[0, :][None, :], axis=1, keepdims=True)
    ad1 = jnp.sum(wx1 * adst[1, :][None, :], axis=1, keepdims=True)
    a_s = jnp.concatenate([as0, as1], axis=1)
    a_d = jnp.concatenate([ad0, ad1], axis=1)
    e = a_s + a_d
    es_o[...] = jnp.exp(jnp.maximum(e, 0.2 * e))
    as_o[...] = a_s
    ad_o[...] = a_d
    y0_o[...] = jnp.concatenate([wx0[:, :32], wx1[:, :32]], axis=1)
    y1_o[...] = jnp.concatenate([wx0[:, 32:], wx1[:, 32:]], axis=1)


def _tc1(x16, aggp, wg, bg, wgat, asrc, adst):
    full = lambda *shape: pl.BlockSpec(shape, lambda i: (0,) * len(shape))
    return pl.pallas_call(
        _tc1_body,
        grid=(_GRID,),
        in_specs=[
            pl.BlockSpec((BR, 16), lambda i: (i, 0)),
            pl.BlockSpec((NC, BR, 16), lambda i: (0, i, 0)),
            full(16, 64), full(64,), full(64, 128), full(2, 64), full(2, 64),
        ],
        out_specs=[
            pl.BlockSpec((BR, 64), lambda i: (i, 0)),
            pl.BlockSpec((BR, 64), lambda i: (i, 0)),
            pl.BlockSpec((BR, 2), lambda i: (i, 0)),
            pl.BlockSpec((BR, 2), lambda i: (i, 0)),
            pl.BlockSpec((BR, 2), lambda i: (i, 0)),
        ],
        out_shape=[
            jax.ShapeDtypeStruct((NP, 64), jnp.float32),
            jax.ShapeDtypeStruct((NP, 64), jnp.float32),
            jax.ShapeDtypeStruct((NP, 2), jnp.float32),
            jax.ShapeDtypeStruct((NP, 2), jnp.float32),
            jax.ShapeDtypeStruct((NP, 2), jnp.float32),
        ],
    )(x16, aggp, wg, bg, wgat, asrc, adst)


def _tcden_body(denp, es, den_o):
    den_o[...] = denp[0] + denp[1] + es[...]


def _tcden(denp, es):
    return pl.pallas_call(
        _tcden_body,
        grid=(_GRID,),
        in_specs=[
            pl.BlockSpec((NC, BR, 2), lambda i: (0, i, 0)),
            pl.BlockSpec((BR, 2), lambda i: (i, 0)),
        ],
        out_specs=pl.BlockSpec((BR, 2), lambda i: (i, 0)),
        out_shape=jax.ShapeDtypeStruct((NP, 2), jnp.float32),
    )(denp, es)


def _tc2_body(nump, den, es, y0, y1, bgat, h2a_o, h2b_o):
    num = jnp.concatenate([nump[0], nump[1]], axis=1)
    wx_h0 = jnp.concatenate([y0[:, :32], y1[:, :32]], axis=1)
    wx_h1 = jnp.concatenate([y0[:, 32:], y1[:, 32:]], axis=1)
    s0 = es[:, 0:1] / den[:, 0:1]
    s1 = es[:, 1:2] / den[:, 1:2]
    h2 = _gelu(0.5 * (num + s0 * wx_h0 + s1 * wx_h1) + bgat[...])
    h2a_o[...] = h2[:, :32]
    h2b_o[...] = h2[:, 32:]


def _tc2(nump, den, es, y0, y1, bgat):
    return pl.pallas_call(
        _tc2_body,
        grid=(_GRID,),
        in_specs=[
            pl.BlockSpec((NC, BR, 32), lambda i: (0, i, 0)),
            pl.BlockSpec((BR, 2), lambda i: (i, 0)),
            pl.BlockSpec((BR, 2), lambda i: (i, 0)),
            pl.BlockSpec((BR, 64), lambda i: (i, 0)),
            pl.BlockSpec((BR, 64), lambda i: (i, 0)),
            pl.BlockSpec((64,), lambda i: (0,)),
        ],
        out_specs=[
            pl.BlockSpec((BR, 32), lambda i: (i, 0)),
            pl.BlockSpec((BR, 32), lambda i: (i, 0)),
        ],
        out_shape=[
            jax.ShapeDtypeStruct((NP, 32), jnp.float32),
            jax.ShapeDtypeStruct((NP, 32), jnp.float32),
        ],
    )(nump, den, es, y0, y1, bgat)


def _tc3_body(aggp, h2a, h2b, wrel, brel, wroot, h3_o):
    agg3 = jnp.concatenate([aggp[0], aggp[1]], axis=1)
    h2 = jnp.concatenate([h2a[...], h2b[...]], axis=1)
    h3_o[...] = _gelu(agg3 @ wrel[...] + brel[...] + h2 @ wroot[...])


def _tc3(aggp, h2a, h2b, wrel, brel, wroot):
    full = lambda *shape: pl.BlockSpec(shape, lambda i: (0,) * len(shape))
    return pl.pallas_call(
        _tc3_body,
        grid=(_GRID,),
        in_specs=[
            pl.BlockSpec((NC, BR, 32), lambda i: (0, i, 0)),
            pl.BlockSpec((BR, 32), lambda i: (i, 0)),
            pl.BlockSpec((BR, 32), lambda i: (i, 0)),
            full(64, 64), full(64,), full(64, 64),
        ],
        out_specs=pl.BlockSpec((BR, 64), lambda i: (i, 0)),
        out_shape=jax.ShapeDtypeStruct((NP, 64), jnp.float32),
    )(aggp, h2a, h2b, wrel, brel, wroot)


def _tc4_body(sum_p, max_p, cnt_p, desc, wsh, bsh, wtg1, btg1, wtg2, btg2,
              wtm1, btm1, wtm2, btm2, out_o, shared_o):
    sums = jnp.sum(sum_p[:, :B, :], axis=0)
    maxs = jnp.max(max_p[:, :B, :], axis=0)
    cnt = jnp.sum(cnt_p[:, :B, 0], axis=0)[:, None]
    mean_p = sums / jnp.maximum(cnt, 1.0)
    maxv = jnp.where(cnt > 0, maxs, 0.0)
    combined = jnp.concatenate([maxv, mean_p, desc[...]], axis=1)
    shared = _gelu(combined @ wsh[...] + bsh[...])
    tg = _gelu(shared @ wtg1[...] + btg1[...]) @ wtg2[...] + btg2[...]
    tm = _gelu(shared @ wtm1[...] + btm1[...]) @ wtm2[...] + btm2[...]
    out_o[...] = jnp.concatenate([tg, tm], axis=1)
    shared_o[...] = shared


def _tc4(sum_p, max_p, cnt_p, desc, wsh, bsh, wtg1, btg1, wtg2, btg2,
         wtm1, btm1, wtm2, btm2):
    return pl.pallas_call(
        _tc4_body,
        out_shape=[
            jax.ShapeDtypeStruct((B, 2), jnp.float32),
            jax.ShapeDtypeStruct((B, 128), jnp.float32),
        ],
    )(sum_p, max_p, cnt_p, desc, wsh, bsh, wtg1, btg1, wtg2, btg2,
      wtm1, btm1, wtm2, btm2)


_seg_sum_gin = _make_seg_sum(16, split_by_worker=True, CK=C_GIN)


def kernel(x, edge_index, batch_index, descriptors, W_gin, b_gin, W_gat,
           att_src, att_dst, b_gat, W_rel, b_rel, W_root, W_sh, b_sh,
           W_tg1, b_tg1, W_tg2, b_tg2, W_tm1, b_tm1, W_tm2, b_tm2):
    src = edge_index[0]
    dst = edge_index[1]
    pad_e = jnp.full((EP - E,), N, jnp.int32)
    srcf = jnp.concatenate([src, pad_e])
    dstf = jnp.concatenate([dst, pad_e])
    srcr = srcf.reshape(NCHUNKS, C)
    dstr = dstf.reshape(NCHUNKS, C)
    src_g = srcf.reshape(EP // C_GIN, C_GIN)
    dst_g = dstf.reshape(EP // C_GIN, C_GIN)
    src_a = srcf.reshape(EP // C_SCA, C_SCA)
    dst_a = dstf.reshape(EP // C_SCA, C_SCA)
    src_c = srcf.reshape(EP // C_GCV, C_GCV)
    dst_c = dstf.reshape(EP // C_GCV, C_GCV)
    x16 = jnp.zeros((NP, 16), jnp.float32).at[:N, :9].set(x)
    bpad = jnp.concatenate(
        [batch_index, jnp.full((NP - N,), B, jnp.int32)])
    wg16 = jnp.zeros((16, 64), jnp.float32).at[:9].set(W_gin)

    aggp = _seg_sum_gin(x16, src_g, dst_g)
    y0, y1, a_s, a_d, es = _tc1(x16, aggp, wg16, b_gin, W_gat,
                                att_src, att_dst)
    ee, denp = _gat_edge(a_s, a_d, src_a, dst_a)
    den = _tcden(denp, es)
    nump = _gat_msg(y0, y1, ee, den, srcr, dstr)
    h2a, h2b = _tc2(nump, den, es, y0, y1, b_gat)
    agg3p = _gconv(h2a, h2b, src_c, dst_c)
    h3 = _tc3(agg3p, h2a, h2b, W_rel, b_rel, W_root)
    sum_p, max_p, cnt_p = _pool(h3, bpad)
    out, shared = _tc4(sum_p, max_p, cnt_p, descriptors, W_sh, b_sh,
                       W_tg1, b_tg1, W_tg2, b_tg2, W_tm1, b_tm1,
                       W_tm2, b_tm2)
    return out, shared


# speedup vs baseline: 1.1682x; 1.1682x over previous
import functools

import jax
import jax.numpy as jnp
from jax import lax
from jax.experimental import pallas as pl
from jax.experimental.pallas import tpu as pltpu
from jax.experimental.pallas import tpu_sc as plsc

N, E, B, H, D = 50000, 800000, 64, 64, 128
NC, NS, L = 2, 16, 16
NW = NC * NS
C = 112
C_GIN = 512
C_SCA = 512
C_GCV = 256
NP = 50176
EP = 802816
NCHUNKS = EP // C
CH_W = NCHUNKS // NW
CH_S = NCHUNKS // NS
RPS = NP // NS
PN_W = NP // NW
PCH = 392
BP = B + 1

_mesh = plsc.VectorSubcoreMesh(core_axis_name="c", subcore_axis_name="s")


def _iota16():
    return lax.iota(jnp.int32, 16)


def _fill2d(ref, rows, cols, value):
    it = _iota16()
    v = jnp.full((16,), value, jnp.float32)
    ngroups = rows * cols // 16

    @pl.loop(0, ngroups)
    def _(g):
        flat = g * 16 + it
        plsc.store_scatter(ref, [flat // cols, flat % cols], v)


ZR = 16


def _zero_spmem(zbuf, accum, s, rows, cols):
    _fill2d(zbuf, ZR, cols, 0.0)
    nblk = rows // ZR

    @pl.loop(0, nblk)
    def _(t):
        pltpu.sync_copy(zbuf, accum.at[pl.ds(s * rows + t * ZR, ZR)])


def _make_seg_sum(F, split_by_worker, CK):
    nch = EP // CK // (NW if split_by_worker else NS)

    def body(table, srcr, dstr, out, idx_s, idx_d, rows, zbuf, accum,
             sem0, sem1):
        c = lax.axis_index("c")
        s = lax.axis_index("s")
        _zero_spmem(zbuf, accum, s, RPS, F)
        plsc.subcore_barrier()
        if split_by_worker:
            base = (c * NS + s) * nch
        else:
            base = s * nch
        sems = (sem0, sem1)

        def prep(slot, j, sem):
            pltpu.sync_copy(srcr.at[base + j], idx_s.at[slot])
            pltpu.sync_copy(dstr.at[base + j], idx_d.at[slot])
            return pltpu.async_copy(table.at[idx_s.at[slot]], rows.at[slot],
                                    sem)

        def drain(slot, sem):
            pltpu.make_async_copy(table.at[idx_s.at[slot]], rows.at[slot],
                                  sem).wait()
            pltpu.sync_copy(rows.at[slot], accum.at[idx_d.at[slot]], add=True)

        prep(0, 0, sem0)

        @pl.loop(0, nch // 2)
        def _(t):
            prep(1, 2 * t + 1, sem1)
            drain(0, sem0)

            @pl.when(t < nch // 2 - 1)
            def _():
                prep(0, 2 * t + 2, sem0)

            drain(1, sem1)

        plsc.subcore_barrier()
        pltpu.sync_copy(accum.at[pl.ds(s * RPS, RPS)],
                        out.at[c, pl.ds(s * RPS, RPS)])

    return pl.kernel(
        body,
        out_type=jax.ShapeDtypeStruct((NC, NP, F), jnp.float32),
        mesh=_mesh,
        compiler_params=pltpu.CompilerParams(use_tc_tiling_on_sc=False, needs_layout_passes=False),
        scratch_types=[
            pltpu.VMEM((2, CK), jnp.int32),
            pltpu.VMEM((2, CK), jnp.int32),
            pltpu.VMEM((2, CK, F), jnp.float32),
            pltpu.VMEM((ZR, F), jnp.float32),
            pltpu.VMEM_SHARED((NP, F), jnp.float32),
            pltpu.SemaphoreType.DMA,
            pltpu.SemaphoreType.DMA,
        ],
    )


def _gat_edge_body(a_src, a_dst, srcr, dstr, ee_out, den_out,
                   idx_s, idx_d, abuf_s, abuf_d, eebuf, zbuf, accum,
                   sem0, sem1):
    c = lax.axis_index("c")
    s = lax.axis_index("s")
    _zero_spmem(zbuf, accum, s, RPS, 2)
    plsc.subcore_barrier()
    base = (c * NS + s) * (EP // C_SCA // NW)
    it = _iota16()
    row_h = it // 2
    col_h = it % 2

    def prep(slot, j, sem):
        pltpu.sync_copy(srcr.at[base + j], idx_s.at[slot])
        pltpu.sync_copy(dstr.at[base + j], idx_d.at[slot])
        pltpu.async_copy(a_src.at[idx_s.at[slot]], abuf_s.at[slot], sem)
        pltpu.async_copy(a_dst.at[idx_d.at[slot]], abuf_d.at[slot], sem)

    def drain(slot, j, sem):
        pltpu.make_async_copy(a_src.at[idx_s.at[slot]], abuf_s.at[slot],
                              sem).wait()
        pltpu.make_async_copy(a_dst.at[idx_d.at[slot]], abuf_d.at[slot],
                              sem).wait()

        @pl.loop(0, 2 * C_SCA // 16, unroll=4)
        def _(g):
            r = row_h + g * 8
            vs = plsc.load_gather(abuf_s.at[slot], [r, col_h])
            vd = plsc.load_gather(abuf_d.at[slot], [r, col_h])
            e = vs + vd
            e = jnp.maximum(e, 0.2 * e)
            plsc.store_scatter(eebuf.at[slot], [r, col_h], jnp.exp(e))

        pltpu.sync_copy(eebuf.at[slot], accum.at[idx_d.at[slot]], add=True)
        pltpu.sync_copy(eebuf.at[slot],
                        ee_out.at[pl.ds((base + j) * C_SCA, C_SCA)])

    nch_a = EP // C_SCA // NW
    prep(0, 0, sem0)

    @pl.loop(0, nch_a // 2)
    def _(t):
        prep(1, 2 * t + 1, sem1)
        drain(0, 2 * t, sem0)

        @pl.when(t < nch_a // 2 - 1)
        def _():
            prep(0, 2 * t + 2, sem0)

        drain(1, 2 * t + 1, sem1)

    plsc.subcore_barrier()
    pltpu.sync_copy(accum.at[pl.ds(s * RPS, RPS)],
                    den_out.at[c, pl.ds(s * RPS, RPS)])


_gat_edge = pl.kernel(
    _gat_edge_body,
    out_type=(jax.ShapeDtypeStruct((EP, 2), jnp.float32),
              jax.ShapeDtypeStruct((NC, NP, 2), jnp.float32)),
    mesh=_mesh,
    compiler_params=pltpu.CompilerParams(use_tc_tiling_on_sc=False, needs_layout_passes=False),
    scratch_types=[
        pltpu.VMEM((2, C_SCA), jnp.int32),
        pltpu.VMEM((2, C_SCA), jnp.int32),
        pltpu.VMEM((2, C_SCA, 2), jnp.float32),
        pltpu.VMEM((2, C_SCA, 2), jnp.float32),
        pltpu.VMEM((2, C_SCA, 2), jnp.float32),
        pltpu.VMEM((ZR, 2), jnp.float32),
        pltpu.VMEM_SHARED((NP, 2), jnp.float32),
        pltpu.SemaphoreType.DMA,
        pltpu.SemaphoreType.DMA,
    ],
)


def _gat_msg_body(y0, y1, ee, den, srcr, dstr, out,
                  idx_s, idx_d, idx_sc, rows, eebuf, denbuf, wbuf, msg, zbuf,
                  accum, sem0, sem1, sem2, sem3):
    c = lax.axis_index("c")
    s = lax.axis_index("s")
    _zero_spmem(zbuf, accum, s, RPS, 32)
    plsc.subcore_barrier()
    base = s * CH_S
    it = _iota16()
    row_h = it // 2
    col_h = it % 2
    gsems = (sem0, sem1)
    ssems = (sem2, sem3)

    def prep(slot, j):
        pltpu.sync_copy(srcr.at[base + j], idx_s.at[slot])
        pltpu.sync_copy(dstr.at[base + j], idx_d.at[slot])

        @pl.when(c == 0)
        def _():
            pltpu.async_copy(y0.at[idx_s.at[slot]], rows.at[slot],
                             gsems[slot])

        @pl.when(c == 1)
        def _():
            pltpu.async_copy(y1.at[idx_s.at[slot]], rows.at[slot],
                             gsems[slot])

        pltpu.async_copy(den.at[idx_d.at[slot]], denbuf.at[slot],
                         gsems[slot])
        pltpu.sync_copy(ee.at[pl.ds((base + j) * C, C)], eebuf.at[slot])

    def drain(slot, t):
        pltpu.make_async_copy(y0.at[idx_s.at[slot]], rows.at[slot],
                              gsems[slot]).wait()
        pltpu.make_async_copy(den.at[idx_d.at[slot]], denbuf.at[slot],
                              gsems[slot]).wait()

        @pl.when(t > 0)
        def _():
            pltpu.make_async_copy(msg.at[slot], accum.at[idx_sc.at[slot]],
                                  ssems[slot]).wait()

        @pl.loop(0, 2 * C // 16, unroll=4)
        def _(g):
            r = row_h + g * 8
            ve = plsc.load_gather(eebuf.at[slot], [r, col_h])
            vd = plsc.load_gather(denbuf.at[slot], [r, col_h])
            plsc.store_scatter(wbuf, [r, col_h], ve / vd)

        @pl.loop(0, C, unroll=8)
        def _(k):
            vk = jnp.full((16,), k, jnp.int32)
            z = jnp.zeros((16,), jnp.int32)
            e0 = plsc.load_gather(wbuf, [vk, z])
            e1 = plsc.load_gather(wbuf, [vk, z + 1])
            r0 = rows[slot, k, pl.ds(0, 16)]
            r1 = rows[slot, k, pl.ds(16, 16)]
            r2 = rows[slot, k, pl.ds(32, 16)]
            r3 = rows[slot, k, pl.ds(48, 16)]
            msg[slot, k, pl.ds(0, 16)] = r0 * e0 + r2 * e1
            msg[slot, k, pl.ds(16, 16)] = r1 * e0 + r3 * e1

        @pl.loop(0, C // 16, unroll=8)
        def _(g):
            idx_sc[slot, pl.ds(g * 16, 16)] = idx_d[slot, pl.ds(g * 16, 16)]

        pltpu.async_copy(msg.at[slot], accum.at[idx_sc.at[slot]],
                         ssems[slot], add=True)

    prep(0, 0)

    @pl.loop(0, CH_S // 2)
    def _(t):
        prep(1, 2 * t + 1)
        drain(0, t)

        @pl.when(t < CH_S // 2 - 1)
        def _():
            prep(0, 2 * t + 2)

        drain(1, t)

    pltpu.make_async_copy(msg.at[0], accum.at[idx_sc.at[0]], sem2).wait()
    pltpu.make_async_copy(msg.at[1], accum.at[idx_sc.at[1]], sem3).wait()
    plsc.subcore_barrier()
    pltpu.sync_copy(accum.at[pl.ds(s * RPS, RPS)],
                    out.at[c, pl.ds(s * RPS, RPS)])


_gat_msg = pl.kernel(
    _gat_msg_body,
    out_type=jax.ShapeDtypeStruct((NC, NP, 32), jnp.float32),
    mesh=_mesh,
    compiler_params=pltpu.CompilerParams(use_tc_tiling_on_sc=False, needs_layout_passes=False),
    scratch_types=[
        pltpu.VMEM((2, C), jnp.int32),
        pltpu.VMEM((2, C), jnp.int32),
        pltpu.VMEM((2, C), jnp.int32),
        pltpu.VMEM((2, C, 64), jnp.float32),
        pltpu.VMEM((2, C, 2), jnp.float32),
        pltpu.VMEM((2, C, 2), jnp.float32),
        pltpu.VMEM((C, 2), jnp.float32),
        pltpu.VMEM((2, C, 32), jnp.float32),
        pltpu.VMEM((ZR, 32), jnp.float32),
        pltpu.VMEM_SHARED((NP, 32), jnp.float32),
        pltpu.SemaphoreType.DMA,
        pltpu.SemaphoreType.DMA,
        pltpu.SemaphoreType.DMA,
        pltpu.SemaphoreType.DMA,
    ],
)


def _gconv_body(h2a, h2b, srcr, dstr, out,
                idx_s, idx_d, rows, zbuf, accum, sem0, sem1):
    c = lax.axis_index("c")
    s = lax.axis_index("s")
    _zero_spmem(zbuf, accum, s, RPS, 32)
    plsc.subcore_barrier()
    nch_g = EP // C_GCV // NS
    base = s * nch_g

    def prep(slot, j, sem):
        pltpu.sync_copy(srcr.at[base + j], idx_s.at[slot])
        pltpu.sync_copy(dstr.at[base + j], idx_d.at[slot])

        @pl.when(c == 0)
        def _():
            pltpu.async_copy(h2a.at[idx_s.at[slot]], rows.at[slot], sem)

        @pl.when(c == 1)
        def _():
            pltpu.async_copy(h2b.at[idx_s.at[slot]], rows.at[slot], sem)

    def drain(slot, sem):
        pltpu.make_async_copy(h2a.at[idx_s.at[slot]], rows.at[slot],
                              sem).wait()
        pltpu.sync_copy(rows.at[slot], accum.at[idx_d.at[slot]], add=True)

    prep(0, 0, sem0)

    @pl.loop(0, nch_g // 2)
    def _(t):
        prep(1, 2 * t + 1, sem1)
        drain(0, sem0)

        @pl.when(t < nch_g // 2 - 1)
        def _():
            prep(0, 2 * t + 2, sem0)

        drain(1, sem1)

    plsc.subcore_barrier()
    pltpu.sync_copy(accum.at[pl.ds(s * RPS, RPS)],
                    out.at[c, pl.ds(s * RPS, RPS)])


_gconv = pl.kernel(
    _gconv_body,
    out_type=jax.ShapeDtypeStruct((NC, NP, 32), jnp.float32),
    mesh=_mesh,
    compiler_params=pltpu.CompilerParams(use_tc_tiling_on_sc=False, needs_layout_passes=False),
    scratch_types=[
        pltpu.VMEM((2, C_GCV), jnp.int32),
        pltpu.VMEM((2, C_GCV), jnp.int32),
        pltpu.VMEM((2, C_GCV, 32), jnp.float32),
        pltpu.VMEM((ZR, 32), jnp.float32),
        pltpu.VMEM_SHARED((NP, 32), jnp.float32),
        pltpu.SemaphoreType.DMA,
        pltpu.SemaphoreType.DMA,
    ],
)


def _pool_body(h3, bidx, sum_out, max_out, cnt_out,
               h3buf, bbuf, sacc, macc, cacc):
    c = lax.axis_index("c")
    s = lax.axis_index("s")
    w = c * NS + s
    _fill2d(sacc, BP, 64, 0.0)
    _fill2d(macc, BP, 64, float("-inf"))
    _fill2d(cacc, BP, 16, 0.0)

    it = _iota16()
    lane0 = it == 0
    ones = jnp.ones((16,), jnp.float32)

    @pl.loop(0, PN_W // PCH)
    def _(t):
        row0 = w * PN_W + t * PCH
        pltpu.sync_copy(h3.at[pl.ds(row0, PCH)], h3buf)
        pltpu.sync_copy(bidx.at[pl.ds(row0, PCH)], bbuf)

        @pl.loop(0, PCH)
        def _(i):
            vi = jnp.full((16,), i, jnp.int32)
            vb = plsc.load_gather(bbuf, [vi])
            plsc.addupdate_scatter(cacc, [vb, it], ones, mask=lane0)
            for k in range(4):
                col = k * 16 + it
                v = plsc.load_gather(h3buf, [vi, col])
                plsc.addupdate_scatter(sacc, [vb, col], v)
                cur = plsc.load_gather(macc, [vb, col])
                plsc.store_scatter(macc, [vb, col], jnp.maximum(cur, v))

    pltpu.sync_copy(sacc, sum_out.at[w])
    pltpu.sync_copy(macc, max_out.at[w])
    pltpu.sync_copy(cacc, cnt_out.at[w])


_pool = pl.kernel(
    _pool_body,
    out_type=(jax.ShapeDtypeStruct((NW, BP, 64), jnp.float32),
              jax.ShapeDtypeStruct((NW, BP, 64), jnp.float32),
              jax.ShapeDtypeStruct((NW, BP, 16), jnp.float32)),
    mesh=_mesh,
    compiler_params=pltpu.CompilerParams(use_tc_tiling_on_sc=False, needs_layout_passes=False),
    scratch_types=[
        pltpu.VMEM((PCH, 64), jnp.float32),
        pltpu.VMEM((PCH,), jnp.int32),
        pltpu.VMEM((BP, 64), jnp.float32),
        pltpu.VMEM((BP, 64), jnp.float32),
        pltpu.VMEM((BP, 16), jnp.float32),
    ],
)


def _gelu(v):
    return 0.5 * v * (1.0 + lax.erf(v * 0.7071067811865476))


BR = 1568
_GRID = NP // BR


def _tc1_body(x16, aggp, wg, bg, wgat, asrc, adst,
              y0_o, y1_o, as_o, ad_o, es_o):
    agg = aggp[0] + aggp[1]
    h = _gelu((x16[...] + agg) @ wg[...] + bg[...])
    wx = h @ wgat[...]
    wx0 = wx[:, :64]
    wx1 = wx[:, 64:]
    as0 = jnp.sum(wx0 * asrc[0, :][None, :], axis=1, keepdims=True)
    as1 = jnp.sum(wx1 * asrc[1, :][None, :], axis=1, keepdims=True)
    ad0 = jnp.sum(wx0 * adst[0, :][None, :], axis=1, keepdims=True)
    ad1 = jnp.sum(wx1 * adst[1, :][None, :], axis=1, keepdims=True)
    a_s = jnp.concatenate([as0, as1], axis=1)
    a_d = jnp.concatenate([ad0, ad1], axis=1)
    e = a_s + a_d
    es_o[...] = jnp.exp(jnp.maximum(e, 0.2 * e))
    as_o[...] = a_s
    ad_o[...] = a_d
    y0_o[...] = jnp.concatenate([wx0[:, :32], wx1[:, :32]], axis=1)
    y1_o[...] = jnp.concatenate([wx0[:, 32:], wx1[:, 32:]], axis=1)


def _tc1(x16, aggp, wg, bg, wgat, asrc, adst):
    full = lambda *shape: pl.BlockSpec(shape, lambda i: (0,) * len(shape))
    return pl.pallas_call(
        _tc1_body,
        grid=(_GRID,),
        in_specs=[
            pl.BlockSpec((BR, 16), lambda i: (i, 0)),
            pl.BlockSpec((NC, BR, 16), lambda i: (0, i, 0)),
            full(16, 64), full(64,), full(64, 128), full(2, 64), full(2, 64),
        ],
        out_specs=[
            pl.BlockSpec((BR, 64), lambda i: (i, 0)),
            pl.BlockSpec((BR, 64), lambda i: (i, 0)),
            pl.BlockSpec((BR, 2), lambda i: (i, 0)),
            pl.BlockSpec((BR, 2), lambda i: (i, 0)),
            pl.BlockSpec((BR, 2), lambda i: (i, 0)),
        ],
        out_shape=[
            jax.ShapeDtypeStruct((NP, 64), jnp.float32),
            jax.ShapeDtypeStruct((NP, 64), jnp.float32),
            jax.ShapeDtypeStruct((NP, 2), jnp.float32),
            jax.ShapeDtypeStruct((NP, 2), jnp.float32),
            jax.ShapeDtypeStruct((NP, 2), jnp.float32),
        ],
    )(x16, aggp, wg, bg, wgat, asrc, adst)


def _tcden_body(denp, es, den_o):
    den_o[...] = denp[0] + denp[1] + es[...]


def _tcden(denp, es):
    return pl.pallas_call(
        _tcden_body,
        grid=(_GRID,),
        in_specs=[
            pl.BlockSpec((NC, BR, 2), lambda i: (0, i, 0)),
            pl.BlockSpec((BR, 2), lambda i: (i, 0)),
        ],
        out_specs=pl.BlockSpec((BR, 2), lambda i: (i, 0)),
        out_shape=jax.ShapeDtypeStruct((NP, 2), jnp.float32),
    )(denp, es)


def _tc2_body(nump, den, es, y0, y1, bgat, h2a_o, h2b_o):
    num = jnp.concatenate([nump[0], nump[1]], axis=1)
    wx_h0 = jnp.concatenate([y0[:, :32], y1[:, :32]], axis=1)
    wx_h1 = jnp.concatenate([y0[:, 32:], y1[:, 32:]], axis=1)
    s0 = es[:, 0:1] / den[:, 0:1]
    s1 = es[:, 1:2] / den[:, 1:2]
    h2 = _gelu(0.5 * (num + s0 * wx_h0 + s1 * wx_h1) + bgat[...])
    h2a_o[...] = h2[:, :32]
    h2b_o[...] = h2[:, 32:]


def _tc2(nump, den, es, y0, y1, bgat):
    return pl.pallas_call(
        _tc2_body,
        grid=(_GRID,),
        in_specs=[
            pl.BlockSpec((NC, BR, 32), lambda i: (0, i, 0)),
            pl.BlockSpec((BR, 2), lambda i: (i, 0)),
            pl.BlockSpec((BR, 2), lambda i: (i, 0)),
            pl.BlockSpec((BR, 64), lambda i: (i, 0)),
            pl.BlockSpec((BR, 64), lambda i: (i, 0)),
            pl.BlockSpec((64,), lambda i: (0,)),
        ],
        out_specs=[
            pl.BlockSpec((BR, 32), lambda i: (i, 0)),
            pl.BlockSpec((BR, 32), lambda i: (i, 0)),
        ],
        out_shape=[
            jax.ShapeDtypeStruct((NP, 32), jnp.float32),
            jax.ShapeDtypeStruct((NP, 32), jnp.float32),
        ],
    )(nump, den, es, y0, y1, bgat)


def _tc3_body(aggp, h2a, h2b, wrel, brel, wroot, h3_o):
    agg3 = jnp.concatenate([aggp[0], aggp[1]], axis=1)
    h2 = jnp.concatenate([h2a[...], h2b[...]], axis=1)
    h3_o[...] = _gelu(agg3 @ wrel[...] + brel[...] + h2 @ wroot[...])


def _tc3(aggp, h2a, h2b, wrel, brel, wroot):
    full = lambda *shape: pl.BlockSpec(shape, lambda i: (0,) * len(shape))
    return pl.pallas_call(
        _tc3_body,
        grid=(_GRID,),
        in_specs=[
            pl.BlockSpec((NC, BR, 32), lambda i: (0, i, 0)),
            pl.BlockSpec((BR, 32), lambda i: (i, 0)),
            pl.BlockSpec((BR, 32), lambda i: (i, 0)),
            full(64, 64), full(64,), full(64, 64),
        ],
        out_specs=pl.BlockSpec((BR, 64), lambda i: (i, 0)),
        out_shape=jax.ShapeDtypeStruct((NP, 64), jnp.float32),
    )(aggp, h2a, h2b, wrel, brel, wroot)


def _tc4_body(sum_p, max_p, cnt_p, desc, wsh, bsh, wtg1, btg1, wtg2, btg2,
              wtm1, btm1, wtm2, btm2, out_o, shared_o):
    sums = jnp.sum(sum_p[:, :B, :], axis=0)
    maxs = jnp.max(max_p[:, :B, :], axis=0)
    cnt = jnp.sum(cnt_p[:, :B, 0], axis=0)[:, None]
    mean_p = sums / jnp.maximum(cnt, 1.0)
    maxv = jnp.where(cnt > 0, maxs, 0.0)
    combined = jnp.concatenate([maxv, mean_p, desc[...]], axis=1)
    shared = _gelu(combined @ wsh[...] + bsh[...])
    tg = _gelu(shared @ wtg1[...] + btg1[...]) @ wtg2[...] + btg2[...]
    tm = _gelu(shared @ wtm1[...] + btm1[...]) @ wtm2[...] + btm2[...]
    out_o[...] = jnp.concatenate([tg, tm], axis=1)
    shared_o[...] = shared


def _tc4(sum_p, max_p, cnt_p, desc, wsh, bsh, wtg1, btg1, wtg2, btg2,
         wtm1, btm1, wtm2, btm2):
    return pl.pallas_call(
        _tc4_body,
        out_shape=[
            jax.ShapeDtypeStruct((B, 2), jnp.float32),
            jax.ShapeDtypeStruct((B, 128), jnp.float32),
        ],
    )(sum_p, max_p, cnt_p, desc, wsh, bsh, wtg1, btg1, wtg2, btg2,
      wtm1, btm1, wtm2, btm2)


_seg_sum_gin = _make_seg_sum(16, split_by_worker=True, CK=C_GIN)


def kernel(x, edge_index, batch_index, descriptors, W_gin, b_gin, W_gat,
           att_src, att_dst, b_gat, W_rel, b_rel, W_root, W_sh, b_sh,
           W_tg1, b_tg1, W_tg2, b_tg2, W_tm1, b_tm1, W_tm2, b_tm2):
    src = edge_index[0]
    dst = edge_index[1]
    pad_e = jnp.full((EP - E,), N, jnp.int32)
    srcf = jnp.concatenate([src, pad_e])
    dstf = jnp.concatenate([dst, pad_e])
    srcr = srcf.reshape(NCHUNKS, C)
    dstr = dstf.reshape(NCHUNKS, C)
    src_g = srcf.reshape(EP // C_GIN, C_GIN)
    dst_g = dstf.reshape(EP // C_GIN, C_GIN)
    src_a = srcf.reshape(EP // C_SCA, C_SCA)
    dst_a = dstf.reshape(EP // C_SCA, C_SCA)
    src_c = srcf.reshape(EP // C_GCV, C_GCV)
    dst_c = dstf.reshape(EP // C_GCV, C_GCV)
    x16 = jnp.zeros((NP, 16), jnp.float32).at[:N, :9].set(x)
    bpad = jnp.concatenate(
        [batch_index, jnp.full((NP - N,), B, jnp.int32)])
    wg16 = jnp.zeros((16, 64), jnp.float32).at[:9].set(W_gin)

    aggp = _seg_sum_gin(x16, src_g, dst_g)
    y0, y1, a_s, a_d, es = _tc1(x16, aggp, wg16, b_gin, W_gat,
                                att_src, att_dst)
    ee, denp = _gat_edge(a_s, a_d, src_a, dst_a)
    den = _tcden(denp, es)
    nump = _gat_msg(y0, y1, ee, den, srcr, dstr)
    h2a, h2b = _tc2(nump, den, es, y0, y1, b_gat)
    agg3p = _gconv(h2a, h2b, src_c, dst_c)
    h3 = _tc3(agg3p, h2a, h2b, W_rel, b_rel, W_root)
    sum_p, max_p, cnt_p = _pool(h3, bpad)
    out, shared = _tc4(sum_p, max_p, cnt_p, descriptors, W_sh, b_sh,
                       W_tg1, b_tg1, W_tg2, b_tg2, W_tm1, b_tm1,
                       W_tm2, b_tm2)
    return out, shared


# speedup vs baseline: 1.1966x; 1.0243x over previous
import functools

import jax
import jax.numpy as jnp
from jax import lax
from jax.experimental import pallas as pl
from jax.experimental.pallas import tpu as pltpu
from jax.experimental.pallas import tpu_sc as plsc

N, E, B, H, D = 50000, 800000, 64, 64, 128
NC, NS, L = 2, 16, 16
NW = NC * NS
C = 112
C_GIN = 512
C_SCA = 512
C_GCV = 256
NP = 50176
EP = 802816
NCHUNKS = EP // C
CH_W = NCHUNKS // NW
CH_S = NCHUNKS // NS
RPS = NP // NS
PN_W = NP // NW
PCH = 392
BP = B + 1

_mesh = plsc.VectorSubcoreMesh(core_axis_name="c", subcore_axis_name="s")


def _iota16():
    return lax.iota(jnp.int32, 16)


def _fill2d(ref, rows, cols, value):
    it = _iota16()
    v = jnp.full((16,), value, jnp.float32)
    ngroups = rows * cols // 16

    @pl.loop(0, ngroups)
    def _(g):
        flat = g * 16 + it
        plsc.store_scatter(ref, [flat // cols, flat % cols], v)


ZR = 16


def _zero_spmem(zbuf, accum, s, rows, cols):
    _fill2d(zbuf, ZR, cols, 0.0)
    nblk = rows // ZR

    @pl.loop(0, nblk)
    def _(t):
        pltpu.sync_copy(zbuf, accum.at[pl.ds(s * rows + t * ZR, ZR)])


def _make_seg_sum(F, split_by_worker, CK):
    nch = EP // CK // (NW if split_by_worker else NS)

    def body(table, srcr, dstr, out, idx_s, idx_d, rows, zbuf, accum,
             sem0, sem1):
        c = lax.axis_index("c")
        s = lax.axis_index("s")
        _zero_spmem(zbuf, accum, s, RPS, F)
        plsc.subcore_barrier()
        if split_by_worker:
            base = (c * NS + s) * nch
        else:
            base = s * nch
        sems = (sem0, sem1)

        def prep(slot, j, sem):
            pltpu.sync_copy(srcr.at[base + j], idx_s.at[slot])
            pltpu.sync_copy(dstr.at[base + j], idx_d.at[slot])
            return pltpu.async_copy(table.at[idx_s.at[slot]], rows.at[slot],
                                    sem)

        def drain(slot, sem):
            pltpu.make_async_copy(table.at[idx_s.at[slot]], rows.at[slot],
                                  sem).wait()
            pltpu.sync_copy(rows.at[slot], accum.at[idx_d.at[slot]], add=True)

        prep(0, 0, sem0)

        @pl.loop(0, nch // 2)
        def _(t):
            prep(1, 2 * t + 1, sem1)
            drain(0, sem0)

            @pl.when(t < nch // 2 - 1)
            def _():
                prep(0, 2 * t + 2, sem0)

            drain(1, sem1)

        plsc.subcore_barrier()
        pltpu.sync_copy(accum.at[pl.ds(s * RPS, RPS)],
                        out.at[c, pl.ds(s * RPS, RPS)])

    return pl.kernel(
        body,
        out_type=jax.ShapeDtypeStruct((NC, NP, F), jnp.float32),
        mesh=_mesh,
        compiler_params=pltpu.CompilerParams(use_tc_tiling_on_sc=False, needs_layout_passes=False),
        scratch_types=[
            pltpu.VMEM((2, CK), jnp.int32),
            pltpu.VMEM((2, CK), jnp.int32),
            pltpu.VMEM((2, CK, F), jnp.float32),
            pltpu.VMEM((ZR, F), jnp.float32),
            pltpu.VMEM_SHARED((NP, F), jnp.float32),
            pltpu.SemaphoreType.DMA,
            pltpu.SemaphoreType.DMA,
        ],
    )


def _gat_edge_body(a_src, a_dst, srcr, dstr, ee_out, den_out,
                   idx_s, idx_d, abuf_s, abuf_d, eebuf, zbuf, accum,
                   sem0, sem1):
    c = lax.axis_index("c")
    s = lax.axis_index("s")
    _zero_spmem(zbuf, accum, s, RPS, 2)
    plsc.subcore_barrier()
    base = (c * NS + s) * (EP // C_SCA // NW)
    it = _iota16()
    row_h = it // 2
    col_h = it % 2

    def prep(slot, j, sem):
        pltpu.sync_copy(srcr.at[base + j], idx_s.at[slot])
        pltpu.sync_copy(dstr.at[base + j], idx_d.at[slot])
        pltpu.async_copy(a_src.at[idx_s.at[slot]], abuf_s.at[slot], sem)
        pltpu.async_copy(a_dst.at[idx_d.at[slot]], abuf_d.at[slot], sem)

    def drain(slot, j, sem):
        pltpu.make_async_copy(a_src.at[idx_s.at[slot]], abuf_s.at[slot],
                              sem).wait()
        pltpu.make_async_copy(a_dst.at[idx_d.at[slot]], abuf_d.at[slot],
                              sem).wait()

        @pl.loop(0, 2 * C_SCA // 16, unroll=4)
        def _(g):
            r = row_h + g * 8
            vs = plsc.load_gather(abuf_s.at[slot], [r, col_h])
            vd = plsc.load_gather(abuf_d.at[slot], [r, col_h])
            e = vs + vd
            e = jnp.maximum(e, 0.2 * e)
            plsc.store_scatter(eebuf.at[slot], [r, col_h], jnp.exp(e))

        pltpu.sync_copy(eebuf.at[slot], accum.at[idx_d.at[slot]], add=True)
        pltpu.sync_copy(eebuf.at[slot],
                        ee_out.at[pl.ds((base + j) * C_SCA, C_SCA)])

    nch_a = EP // C_SCA // NW
    prep(0, 0, sem0)

    @pl.loop(0, nch_a // 2)
    def _(t):
        prep(1, 2 * t + 1, sem1)
        drain(0, 2 * t, sem0)

        @pl.when(t < nch_a // 2 - 1)
        def _():
            prep(0, 2 * t + 2, sem0)

        drain(1, 2 * t + 1, sem1)

    plsc.subcore_barrier()
    pltpu.sync_copy(accum.at[pl.ds(s * RPS, RPS)],
                    den_out.at[c, pl.ds(s * RPS, RPS)])


_gat_edge = pl.kernel(
    _gat_edge_body,
    out_type=(jax.ShapeDtypeStruct((EP, 2), jnp.float32),
              jax.ShapeDtypeStruct((NC, NP, 2), jnp.float32)),
    mesh=_mesh,
    compiler_params=pltpu.CompilerParams(use_tc_tiling_on_sc=False, needs_layout_passes=False),
    scratch_types=[
        pltpu.VMEM((2, C_SCA), jnp.int32),
        pltpu.VMEM((2, C_SCA), jnp.int32),
        pltpu.VMEM((2, C_SCA, 2), jnp.float32),
        pltpu.VMEM((2, C_SCA, 2), jnp.float32),
        pltpu.VMEM((2, C_SCA, 2), jnp.float32),
        pltpu.VMEM((ZR, 2), jnp.float32),
        pltpu.VMEM_SHARED((NP, 2), jnp.float32),
        pltpu.SemaphoreType.DMA,
        pltpu.SemaphoreType.DMA,
    ],
)


def _gat_msg_body(y0, y1, ee, den, srcr, dstr, out,
                  idx_s, idx_d, idx_sc, rows, eebuf, denbuf, wbuf, msg, zbuf,
                  accum, sem0, sem1, sem2, sem3):
    c = lax.axis_index("c")
    s = lax.axis_index("s")
    _zero_spmem(zbuf, accum, s, RPS, 32)
    plsc.subcore_barrier()
    base = s * CH_S
    it = _iota16()
    row_h = it // 2
    col_h = it % 2
    gsems = (sem0, sem1)
    ssems = (sem2, sem3)

    def prep(slot, j):
        pltpu.sync_copy(srcr.at[base + j], idx_s.at[slot])
        pltpu.sync_copy(dstr.at[base + j], idx_d.at[slot])

        @pl.when(c == 0)
        def _():
            pltpu.async_copy(y0.at[idx_s.at[slot]], rows.at[slot],
                             gsems[slot])

        @pl.when(c == 1)
        def _():
            pltpu.async_copy(y1.at[idx_s.at[slot]], rows.at[slot],
                             gsems[slot])

        pltpu.async_copy(den.at[idx_d.at[slot]], denbuf.at[slot],
                         gsems[slot])
        pltpu.sync_copy(ee.at[pl.ds((base + j) * C, C)], eebuf.at[slot])

    def drain(slot, t):
        pltpu.make_async_copy(y0.at[idx_s.at[slot]], rows.at[slot],
                              gsems[slot]).wait()
        pltpu.make_async_copy(den.at[idx_d.at[slot]], denbuf.at[slot],
                              gsems[slot]).wait()

        @pl.when(t > 0)
        def _():
            pltpu.make_async_copy(msg.at[slot], accum.at[idx_sc.at[slot]],
                                  ssems[slot]).wait()

        @pl.loop(0, 2 * C // 16, unroll=4)
        def _(g):
            r = row_h + g * 8
            ve = plsc.load_gather(eebuf.at[slot], [r, col_h])
            vd = plsc.load_gather(denbuf.at[slot], [r, col_h])
            wbuf[pl.ds(g * 16, 16)] = ve / vd

        @pl.loop(0, C // 8, unroll=2)
        def _(g):
            wv = wbuf[pl.ds(g * 16, 16)]
            for l in range(8):
                k = g * 8 + l
                e0 = jnp.full((16,), wv[2 * l])
                e1 = jnp.full((16,), wv[2 * l + 1])
                r0 = rows[slot, k, pl.ds(0, 16)]
                r1 = rows[slot, k, pl.ds(16, 16)]
                r2 = rows[slot, k, pl.ds(32, 16)]
                r3 = rows[slot, k, pl.ds(48, 16)]
                msg[slot, k, pl.ds(0, 16)] = r0 * e0 + r2 * e1
                msg[slot, k, pl.ds(16, 16)] = r1 * e0 + r3 * e1

        @pl.loop(0, C // 16, unroll=8)
        def _(g):
            idx_sc[slot, pl.ds(g * 16, 16)] = idx_d[slot, pl.ds(g * 16, 16)]

        pltpu.async_copy(msg.at[slot], accum.at[idx_sc.at[slot]],
                         ssems[slot], add=True)

    prep(0, 0)

    @pl.loop(0, CH_S // 2)
    def _(t):
        prep(1, 2 * t + 1)
        drain(0, t)

        @pl.when(t < CH_S // 2 - 1)
        def _():
            prep(0, 2 * t + 2)

        drain(1, t)

    pltpu.make_async_copy(msg.at[0], accum.at[idx_sc.at[0]], sem2).wait()
    pltpu.make_async_copy(msg.at[1], accum.at[idx_sc.at[1]], sem3).wait()
    plsc.subcore_barrier()
    pltpu.sync_copy(accum.at[pl.ds(s * RPS, RPS)],
                    out.at[c, pl.ds(s * RPS, RPS)])


_gat_msg = pl.kernel(
    _gat_msg_body,
    out_type=jax.ShapeDtypeStruct((NC, NP, 32), jnp.float32),
    mesh=_mesh,
    compiler_params=pltpu.CompilerParams(use_tc_tiling_on_sc=False, needs_layout_passes=False),
    scratch_types=[
        pltpu.VMEM((2, C), jnp.int32),
        pltpu.VMEM((2, C), jnp.int32),
        pltpu.VMEM((2, C), jnp.int32),
        pltpu.VMEM((2, C, 64), jnp.float32),
        pltpu.VMEM((2, C, 2), jnp.float32),
        pltpu.VMEM((2, C, 2), jnp.float32),
        pltpu.VMEM((2 * C,), jnp.float32),
        pltpu.VMEM((2, C, 32), jnp.float32),
        pltpu.VMEM((ZR, 32), jnp.float32),
        pltpu.VMEM_SHARED((NP, 32), jnp.float32),
        pltpu.SemaphoreType.DMA,
        pltpu.SemaphoreType.DMA,
        pltpu.SemaphoreType.DMA,
        pltpu.SemaphoreType.DMA,
    ],
)


def _gconv_body(h2a, h2b, srcr, dstr, out,
                idx_s, idx_d, rows, zbuf, accum, sem0, sem1):
    c = lax.axis_index("c")
    s = lax.axis_index("s")
    _zero_spmem(zbuf, accum, s, RPS, 32)
    plsc.subcore_barrier()
    nch_g = EP // C_GCV // NS
    base = s * nch_g

    def prep(slot, j, sem):
        pltpu.sync_copy(srcr.at[base + j], idx_s.at[slot])
        pltpu.sync_copy(dstr.at[base + j], idx_d.at[slot])

        @pl.when(c == 0)
        def _():
            pltpu.async_copy(h2a.at[idx_s.at[slot]], rows.at[slot], sem)

        @pl.when(c == 1)
        def _():
            pltpu.async_copy(h2b.at[idx_s.at[slot]], rows.at[slot], sem)

    def drain(slot, sem):
        pltpu.make_async_copy(h2a.at[idx_s.at[slot]], rows.at[slot],
                              sem).wait()
        pltpu.sync_copy(rows.at[slot], accum.at[idx_d.at[slot]], add=True)

    prep(0, 0, sem0)

    @pl.loop(0, nch_g // 2)
    def _(t):
        prep(1, 2 * t + 1, sem1)
        drain(0, sem0)

        @pl.when(t < nch_g // 2 - 1)
        def _():
            prep(0, 2 * t + 2, sem0)

        drain(1, sem1)

    plsc.subcore_barrier()
    pltpu.sync_copy(accum.at[pl.ds(s * RPS, RPS)],
                    out.at[c, pl.ds(s * RPS, RPS)])


_gconv = pl.kernel(
    _gconv_body,
    out_type=jax.ShapeDtypeStruct((NC, NP, 32), jnp.float32),
    mesh=_mesh,
    compiler_params=pltpu.CompilerParams(use_tc_tiling_on_sc=False, needs_layout_passes=False),
    scratch_types=[
        pltpu.VMEM((2, C_GCV), jnp.int32),
        pltpu.VMEM((2, C_GCV), jnp.int32),
        pltpu.VMEM((2, C_GCV, 32), jnp.float32),
        pltpu.VMEM((ZR, 32), jnp.float32),
        pltpu.VMEM_SHARED((NP, 32), jnp.float32),
        pltpu.SemaphoreType.DMA,
        pltpu.SemaphoreType.DMA,
    ],
)


def _pool_body(h3, bidx, sum_out, max_out, cnt_out,
               h3buf, bbuf, sacc, macc, cacc):
    c = lax.axis_index("c")
    s = lax.axis_index("s")
    w = c * NS + s
    _fill2d(sacc, BP, 64, 0.0)
    _fill2d(macc, BP, 64, float("-inf"))
    _fill2d(cacc, BP, 16, 0.0)

    it = _iota16()
    lane0 = it == 0
    ones = jnp.ones((16,), jnp.float32)

    @pl.loop(0, PN_W // PCH)
    def _(t):
        row0 = w * PN_W + t * PCH
        pltpu.sync_copy(h3.at[pl.ds(row0, PCH)], h3buf)
        pltpu.sync_copy(bidx.at[pl.ds(row0, PCH)], bbuf)

        @pl.loop(0, PCH)
        def _(i):
            vi = jnp.full((16,), i, jnp.int32)
            vb = plsc.load_gather(bbuf, [vi])
            plsc.addupdate_scatter(cacc, [vb, it], ones, mask=lane0)
            for k in range(4):
                col = k * 16 + it
                v = plsc.load_gather(h3buf, [vi, col])
                plsc.addupdate_scatter(sacc, [vb, col], v)
                cur = plsc.load_gather(macc, [vb, col])
                plsc.store_scatter(macc, [vb, col], jnp.maximum(cur, v))

    pltpu.sync_copy(sacc, sum_out.at[w])
    pltpu.sync_copy(macc, max_out.at[w])
    pltpu.sync_copy(cacc, cnt_out.at[w])


_pool = pl.kernel(
    _pool_body,
    out_type=(jax.ShapeDtypeStruct((NW, BP, 64), jnp.float32),
              jax.ShapeDtypeStruct((NW, BP, 64), jnp.float32),
              jax.ShapeDtypeStruct((NW, BP, 16), jnp.float32)),
    mesh=_mesh,
    compiler_params=pltpu.CompilerParams(use_tc_tiling_on_sc=False, needs_layout_passes=False),
    scratch_types=[
        pltpu.VMEM((PCH, 64), jnp.float32),
        pltpu.VMEM((PCH,), jnp.int32),
        pltpu.VMEM((BP, 64), jnp.float32),
        pltpu.VMEM((BP, 64), jnp.float32),
        pltpu.VMEM((BP, 16), jnp.float32),
    ],
)


def _gelu(v):
    return 0.5 * v * (1.0 + lax.erf(v * 0.7071067811865476))


BR = 1568
_GRID = NP // BR


def _tc1_body(x16, aggp, wg, bg, wgat, asrc, adst,
              y0_o, y1_o, as_o, ad_o, es_o):
    agg = aggp[0] + aggp[1]
    h = _gelu((x16[...] + agg) @ wg[...] + bg[...])
    wx = h @ wgat[...]
    wx0 = wx[:, :64]
    wx1 = wx[:, 64:]
    as0 = jnp.sum(wx0 * asrc[0, :][None, :], axis=1, keepdims=True)
    as1 = jnp.sum(wx1 * asrc[1, :][None, :], axis=1, keepdims=True)
    ad0 = jnp.sum(wx0 * adst[0, :][None, :], axis=1, keepdims=True)
    ad1 = jnp.sum(wx1 * adst[1, :][None, :], axis=1, keepdims=True)
    a_s = jnp.concatenate([as0, as1], axis=1)
    a_d = jnp.concatenate([ad0, ad1], axis=1)
    e = a_s + a_d
    es_o[...] = jnp.exp(jnp.maximum(e, 0.2 * e))
    as_o[...] = a_s
    ad_o[...] = a_d
    y0_o[...] = jnp.concatenate([wx0[:, :32], wx1[:, :32]], axis=1)
    y1_o[...] = jnp.concatenate([wx0[:, 32:], wx1[:, 32:]], axis=1)


def _tc1(x16, aggp, wg, bg, wgat, asrc, adst):
    full = lambda *shape: pl.BlockSpec(shape, lambda i: (0,) * len(shape))
    return pl.pallas_call(
        _tc1_body,
        grid=(_GRID,),
        in_specs=[
            pl.BlockSpec((BR, 16), lambda i: (i, 0)),
            pl.BlockSpec((NC, BR, 16), lambda i: (0, i, 0)),
            full(16, 64), full(64,), full(64, 128), full(2, 64), full(2, 64),
        ],
        out_specs=[
            pl.BlockSpec((BR, 64), lambda i: (i, 0)),
            pl.BlockSpec((BR, 64), lambda i: (i, 0)),
            pl.BlockSpec((BR, 2), lambda i: (i, 0)),
            pl.BlockSpec((BR, 2), lambda i: (i, 0)),
            pl.BlockSpec((BR, 2), lambda i: (i, 0)),
        ],
        out_shape=[
            jax.ShapeDtypeStruct((NP, 64), jnp.float32),
            jax.ShapeDtypeStruct((NP, 64), jnp.float32),
            jax.ShapeDtypeStruct((NP, 2), jnp.float32),
            jax.ShapeDtypeStruct((NP, 2), jnp.float32),
            jax.ShapeDtypeStruct((NP, 2), jnp.float32),
        ],
    )(x16, aggp, wg, bg, wgat, asrc, adst)


def _tcden_body(denp, es, den_o):
    den_o[...] = denp[0] + denp[1] + es[...]


def _tcden(denp, es):
    return pl.pallas_call(
        _tcden_body,
        grid=(_GRID,),
        in_specs=[
            pl.BlockSpec((NC, BR, 2), lambda i: (0, i, 0)),
            pl.BlockSpec((BR, 2), lambda i: (i, 0)),
        ],
        out_specs=pl.BlockSpec((BR, 2), lambda i: (i, 0)),
        out_shape=jax.ShapeDtypeStruct((NP, 2), jnp.float32),
    )(denp, es)


def _tc2_body(nump, den, es, y0, y1, bgat, h2a_o, h2b_o):
    num = jnp.concatenate([nump[0], nump[1]], axis=1)
    wx_h0 = jnp.concatenate([y0[:, :32], y1[:, :32]], axis=1)
    wx_h1 = jnp.concatenate([y0[:, 32:], y1[:, 32:]], axis=1)
    s0 = es[:, 0:1] / den[:, 0:1]
    s1 = es[:, 1:2] / den[:, 1:2]
    h2 = _gelu(0.5 * (num + s0 * wx_h0 + s1 * wx_h1) + bgat[...])
    h2a_o[...] = h2[:, :32]
    h2b_o[...] = h2[:, 32:]


def _tc2(nump, den, es, y0, y1, bgat):
    return pl.pallas_call(
        _tc2_body,
        grid=(_GRID,),
        in_specs=[
            pl.BlockSpec((NC, BR, 32), lambda i: (0, i, 0)),
            pl.BlockSpec((BR, 2), lambda i: (i, 0)),
            pl.BlockSpec((BR, 2), lambda i: (i, 0)),
            pl.BlockSpec((BR, 64), lambda i: (i, 0)),
            pl.BlockSpec((BR, 64), lambda i: (i, 0)),
            pl.BlockSpec((64,), lambda i: (0,)),
        ],
        out_specs=[
            pl.BlockSpec((BR, 32), lambda i: (i, 0)),
            pl.BlockSpec((BR, 32), lambda i: (i, 0)),
        ],
        out_shape=[
            jax.ShapeDtypeStruct((NP, 32), jnp.float32),
            jax.ShapeDtypeStruct((NP, 32), jnp.float32),
        ],
    )(nump, den, es, y0, y1, bgat)


def _tc3_body(aggp, h2a, h2b, wrel, brel, wroot, h3_o):
    agg3 = jnp.concatenate([aggp[0], aggp[1]], axis=1)
    h2 = jnp.concatenate([h2a[...], h2b[...]], axis=1)
    h3_o[...] = _gelu(agg3 @ wrel[...] + brel[...] + h2 @ wroot[...])


def _tc3(aggp, h2a, h2b, wrel, brel, wroot):
    full = lambda *shape: pl.BlockSpec(shape, lambda i: (0,) * len(shape))
    return pl.pallas_call(
        _tc3_body,
        grid=(_GRID,),
        in_specs=[
            pl.BlockSpec((NC, BR, 32), lambda i: (0, i, 0)),
            pl.BlockSpec((BR, 32), lambda i: (i, 0)),
            pl.BlockSpec((BR, 32), lambda i: (i, 0)),
            full(64, 64), full(64,), full(64, 64),
        ],
        out_specs=pl.BlockSpec((BR, 64), lambda i: (i, 0)),
        out_shape=jax.ShapeDtypeStruct((NP, 64), jnp.float32),
    )(aggp, h2a, h2b, wrel, brel, wroot)


def _tc4_body(sum_p, max_p, cnt_p, desc, wsh, bsh, wtg1, btg1, wtg2, btg2,
              wtm1, btm1, wtm2, btm2, out_o, shared_o):
    sums = jnp.sum(sum_p[:, :B, :], axis=0)
    maxs = jnp.max(max_p[:, :B, :], axis=0)
    cnt = jnp.sum(cnt_p[:, :B, 0], axis=0)[:, None]
    mean_p = sums / jnp.maximum(cnt, 1.0)
    maxv = jnp.where(cnt > 0, maxs, 0.0)
    combined = jnp.concatenate([maxv, mean_p, desc[...]], axis=1)
    shared = _gelu(combined @ wsh[...] + bsh[...])
    tg = _gelu(shared @ wtg1[...] + btg1[...]) @ wtg2[...] + btg2[...]
    tm = _gelu(shared @ wtm1[...] + btm1[...]) @ wtm2[...] + btm2[...]
    out_o[...] = jnp.concatenate([tg, tm], axis=1)
    shared_o[...] = shared


def _tc4(sum_p, max_p, cnt_p, desc, wsh, bsh, wtg1, btg1, wtg2, btg2,
         wtm1, btm1, wtm2, btm2):
    return pl.pallas_call(
        _tc4_body,
        out_shape=[
            jax.ShapeDtypeStruct((B, 2), jnp.float32),
            jax.ShapeDtypeStruct((B, 128), jnp.float32),
        ],
    )(sum_p, max_p, cnt_p, desc, wsh, bsh, wtg1, btg1, wtg2, btg2,
      wtm1, btm1, wtm2, btm2)


_seg_sum_gin = _make_seg_sum(16, split_by_worker=True, CK=C_GIN)


def kernel(x, edge_index, batch_index, descriptors, W_gin, b_gin, W_gat,
           att_src, att_dst, b_gat, W_rel, b_rel, W_root, W_sh, b_sh,
           W_tg1, b_tg1, W_tg2, b_tg2, W_tm1, b_tm1, W_tm2, b_tm2):
    src = edge_index[0]
    dst = edge_index[1]
    pad_e = jnp.full((EP - E,), N, jnp.int32)
    srcf = jnp.concatenate([src, pad_e])
    dstf = jnp.concatenate([dst, pad_e])
    srcr = srcf.reshape(NCHUNKS, C)
    dstr = dstf.reshape(NCHUNKS, C)
    src_g = srcf.reshape(EP // C_GIN, C_GIN)
    dst_g = dstf.reshape(EP // C_GIN, C_GIN)
    src_a = srcf.reshape(EP // C_SCA, C_SCA)
    dst_a = dstf.reshape(EP // C_SCA, C_SCA)
    src_c = srcf.reshape(EP // C_GCV, C_GCV)
    dst_c = dstf.reshape(EP // C_GCV, C_GCV)
    x16 = jnp.zeros((NP, 16), jnp.float32).at[:N, :9].set(x)
    bpad = jnp.concatenate(
        [batch_index, jnp.full((NP - N,), B, jnp.int32)])
    wg16 = jnp.zeros((16, 64), jnp.float32).at[:9].set(W_gin)

    aggp = _seg_sum_gin(x16, src_g, dst_g)
    y0, y1, a_s, a_d, es = _tc1(x16, aggp, wg16, b_gin, W_gat,
                                att_src, att_dst)
    ee, denp = _gat_edge(a_s, a_d, src_a, dst_a)
    den = _tcden(denp, es)
    nump = _gat_msg(y0, y1, ee, den, srcr, dstr)
    h2a, h2b = _tc2(nump, den, es, y0, y1, b_gat)
    agg3p = _gconv(h2a, h2b, src_c, dst_c)
    h3 = _tc3(agg3p, h2a, h2b, W_rel, b_rel, W_root)
    sum_p, max_p, cnt_p = _pool(h3, bpad)
    out, shared = _tc4(sum_p, max_p, cnt_p, descriptors, W_sh, b_sh,
                       W_tg1, b_tg1, W_tg2, b_tg2, W_tm1, b_tm1,
                       W_tm2, b_tm2)
    return out, shared


# speedup vs baseline: 1.3257x; 1.1079x over previous
import functools

import jax
import jax.numpy as jnp
from jax import lax
from jax.experimental import pallas as pl
from jax.experimental.pallas import tpu as pltpu
from jax.experimental.pallas import tpu_sc as plsc

N, E, B, H, D = 50000, 800000, 64, 64, 128
NC, NS, L = 2, 16, 16
NW = NC * NS
C = 112
C_GIN = 512
C_SCA = 512
C_GCV = 256
NP = 50176
EP = 802816
NCHUNKS = EP // C
CH_W = NCHUNKS // NW
CH_S = NCHUNKS // NS
RPS = NP // NS
PN_W = NP // NW
PCH = 392
BP = B + 1

_mesh = plsc.VectorSubcoreMesh(core_axis_name="c", subcore_axis_name="s")


def _iota16():
    return lax.iota(jnp.int32, 16)


def _fill2d(ref, rows, cols, value):
    it = _iota16()
    v = jnp.full((16,), value, jnp.float32)
    ngroups = rows * cols // 16

    @pl.loop(0, ngroups)
    def _(g):
        flat = g * 16 + it
        plsc.store_scatter(ref, [flat // cols, flat % cols], v)


ZR = 16


def _zero_spmem(zbuf, accum, s, rows, cols):
    _fill2d(zbuf, ZR, cols, 0.0)
    nblk = rows // ZR

    @pl.loop(0, nblk)
    def _(t):
        pltpu.sync_copy(zbuf, accum.at[pl.ds(s * rows + t * ZR, ZR)])


def _make_seg_sum(F, split_by_worker, CK):
    nch = EP // CK // (NW if split_by_worker else NS)

    def body(table, srcr, dstr, out, idx_s, idx_d, rows, zbuf, accum,
             sem0, sem1):
        c = lax.axis_index("c")
        s = lax.axis_index("s")
        _zero_spmem(zbuf, accum, s, RPS, F)
        plsc.subcore_barrier()
        if split_by_worker:
            base = (c * NS + s) * nch
        else:
            base = s * nch
        sems = (sem0, sem1)

        def prep(slot, j, sem):
            pltpu.sync_copy(srcr.at[base + j], idx_s.at[slot])
            pltpu.sync_copy(dstr.at[base + j], idx_d.at[slot])
            return pltpu.async_copy(table.at[idx_s.at[slot]], rows.at[slot],
                                    sem)

        def drain(slot, sem):
            pltpu.make_async_copy(table.at[idx_s.at[slot]], rows.at[slot],
                                  sem).wait()
            pltpu.sync_copy(rows.at[slot], accum.at[idx_d.at[slot]], add=True)

        prep(0, 0, sem0)

        @pl.loop(0, nch // 2)
        def _(t):
            prep(1, 2 * t + 1, sem1)
            drain(0, sem0)

            @pl.when(t < nch // 2 - 1)
            def _():
                prep(0, 2 * t + 2, sem0)

            drain(1, sem1)

        plsc.subcore_barrier()
        pltpu.sync_copy(accum.at[pl.ds(s * RPS, RPS)],
                        out.at[c, pl.ds(s * RPS, RPS)])

    return pl.kernel(
        body,
        out_type=jax.ShapeDtypeStruct((NC, NP, F), jnp.float32),
        mesh=_mesh,
        compiler_params=pltpu.CompilerParams(use_tc_tiling_on_sc=False, needs_layout_passes=False),
        scratch_types=[
            pltpu.VMEM((2, CK), jnp.int32),
            pltpu.VMEM((2, CK), jnp.int32),
            pltpu.VMEM((2, CK, F), jnp.float32),
            pltpu.VMEM((ZR, F), jnp.float32),
            pltpu.VMEM_SHARED((NP, F), jnp.float32),
            pltpu.SemaphoreType.DMA,
            pltpu.SemaphoreType.DMA,
        ],
    )


def _gat_edge_body(a_src, a_dst, srcr, dstr, ee_out, den_out,
                   idx_s, idx_d, abuf_s, abuf_d, eebuf, zbuf, accum,
                   sem0, sem1):
    c = lax.axis_index("c")
    s = lax.axis_index("s")
    _zero_spmem(zbuf, accum, s, RPS, 2)
    plsc.subcore_barrier()
    base = (c * NS + s) * (EP // C_SCA // NW)
    it = _iota16()
    row_h = it // 2
    col_h = it % 2

    def prep(slot, j, sem):
        pltpu.sync_copy(srcr.at[base + j], idx_s.at[slot])
        pltpu.sync_copy(dstr.at[base + j], idx_d.at[slot])
        pltpu.async_copy(a_src.at[idx_s.at[slot]], abuf_s.at[slot], sem)
        pltpu.async_copy(a_dst.at[idx_d.at[slot]], abuf_d.at[slot], sem)

    def drain(slot, j, sem):
        pltpu.make_async_copy(a_src.at[idx_s.at[slot]], abuf_s.at[slot],
                              sem).wait()
        pltpu.make_async_copy(a_dst.at[idx_d.at[slot]], abuf_d.at[slot],
                              sem).wait()

        @pl.loop(0, 2 * C_SCA // 16, unroll=4)
        def _(g):
            r = row_h + g * 8
            vs = plsc.load_gather(abuf_s.at[slot], [r, col_h])
            vd = plsc.load_gather(abuf_d.at[slot], [r, col_h])
            e = vs + vd
            e = jnp.maximum(e, 0.2 * e)
            plsc.store_scatter(eebuf.at[slot], [r, col_h], jnp.exp(e))

        pltpu.sync_copy(eebuf.at[slot], accum.at[idx_d.at[slot]], add=True)
        pltpu.sync_copy(eebuf.at[slot],
                        ee_out.at[pl.ds((base + j) * C_SCA, C_SCA)])

    nch_a = EP // C_SCA // NW
    prep(0, 0, sem0)

    @pl.loop(0, nch_a // 2)
    def _(t):
        prep(1, 2 * t + 1, sem1)
        drain(0, 2 * t, sem0)

        @pl.when(t < nch_a // 2 - 1)
        def _():
            prep(0, 2 * t + 2, sem0)

        drain(1, 2 * t + 1, sem1)

    plsc.subcore_barrier()
    pltpu.sync_copy(accum.at[pl.ds(s * RPS, RPS)],
                    den_out.at[c, pl.ds(s * RPS, RPS)])


_gat_edge = pl.kernel(
    _gat_edge_body,
    out_type=(jax.ShapeDtypeStruct((EP, 2), jnp.float32),
              jax.ShapeDtypeStruct((NC, NP, 2), jnp.float32)),
    mesh=_mesh,
    compiler_params=pltpu.CompilerParams(use_tc_tiling_on_sc=False, needs_layout_passes=False),
    scratch_types=[
        pltpu.VMEM((2, C_SCA), jnp.int32),
        pltpu.VMEM((2, C_SCA), jnp.int32),
        pltpu.VMEM((2, C_SCA, 2), jnp.float32),
        pltpu.VMEM((2, C_SCA, 2), jnp.float32),
        pltpu.VMEM((2, C_SCA, 2), jnp.float32),
        pltpu.VMEM((ZR, 2), jnp.float32),
        pltpu.VMEM_SHARED((NP, 2), jnp.float32),
        pltpu.SemaphoreType.DMA,
        pltpu.SemaphoreType.DMA,
    ],
)


def _gat_msg_body(y0, y1, ee, den, z32, srcr, dstr, out,
                  idx_s, idx_d, eebuf, rows, denbuf, wbuf, msg,
                  accum, sem0, sem1, sem2, sem3):
    c = lax.axis_index("c")
    s = lax.axis_index("s")
    pltpu.sync_copy(z32.at[pl.ds(s * RPS, RPS)],
                    accum.at[pl.ds(s * RPS, RPS)])
    plsc.subcore_barrier()
    NBK = 4
    nblk = CH_S // NBK
    base = s * CH_S
    it = _iota16()
    row_h = it // 2
    col_h = it % 2
    gsems = (sem0, sem1)
    ssems = (sem2, sem3)

    def gather(slot, b):
        @pl.when(c == 0)
        def _():
            pltpu.async_copy(y0.at[idx_s.at[b]], rows.at[slot], gsems[slot])

        @pl.when(c == 1)
        def _():
            pltpu.async_copy(y1.at[idx_s.at[b]], rows.at[slot], gsems[slot])

        pltpu.async_copy(den.at[idx_d.at[b]], denbuf.at[slot], gsems[slot])

    def compute(slot, b, do_wait):
        pltpu.make_async_copy(y0.at[idx_s.at[b]], rows.at[slot],
                              gsems[slot]).wait()
        pltpu.make_async_copy(den.at[idx_d.at[b]], denbuf.at[slot],
                              gsems[slot]).wait()

        if do_wait:
            pltpu.make_async_copy(msg.at[slot], accum.at[idx_d.at[b]],
                                  ssems[slot]).wait()

        @pl.loop(0, 2 * C // 16, unroll=4)
        def _(g):
            r = row_h + g * 8
            ve = plsc.load_gather(eebuf, [b * C + r, col_h])
            vd = plsc.load_gather(denbuf.at[slot], [r, col_h])
            wbuf[pl.ds(g * 16, 16)] = ve / vd

        @pl.loop(0, C // 8, unroll=2)
        def _(g):
            wv = wbuf[pl.ds(g * 16, 16)]
            for l in range(8):
                k = g * 8 + l
                e0 = jnp.full((16,), wv[2 * l])
                e1 = jnp.full((16,), wv[2 * l + 1])
                r0 = rows[slot, k, pl.ds(0, 16)]
                r1 = rows[slot, k, pl.ds(16, 16)]
                r2 = rows[slot, k, pl.ds(32, 16)]
                r3 = rows[slot, k, pl.ds(48, 16)]
                msg[slot, k, pl.ds(0, 16)] = r0 * e0 + r2 * e1
                msg[slot, k, pl.ds(16, 16)] = r1 * e0 + r3 * e1

        pltpu.async_copy(msg.at[slot], accum.at[idx_d.at[b]],
                         ssems[slot], add=True)

    @pl.loop(0, nblk)
    def _(t):
        @pl.when(t > 0)
        def _():
            pltpu.make_async_copy(msg.at[0], accum.at[idx_d.at[0]],
                                  sem2).wait()
            pltpu.make_async_copy(msg.at[1], accum.at[idx_d.at[1]],
                                  sem3).wait()

        j0 = base + t * NBK
        pltpu.sync_copy(srcr.at[pl.ds(j0, NBK)], idx_s)
        pltpu.sync_copy(dstr.at[pl.ds(j0, NBK)], idx_d)
        pltpu.sync_copy(ee.at[pl.ds(j0 * C, NBK * C)], eebuf)
        gather(0, 0)
        gather(1, 1)
        for u in range(NBK // 2):
            compute(0, 2 * u, do_wait=(u >= 1))
            if u < NBK // 2 - 1:
                gather(0, 2 * u + 2)
            compute(1, 2 * u + 1, do_wait=(u >= 1))
            if u < NBK // 2 - 1:
                gather(1, 2 * u + 3)

    pltpu.make_async_copy(msg.at[0], accum.at[idx_d.at[0]], sem2).wait()
    pltpu.make_async_copy(msg.at[1], accum.at[idx_d.at[1]], sem3).wait()
    plsc.subcore_barrier()
    pltpu.sync_copy(accum.at[pl.ds(s * RPS, RPS)],
                    out.at[c, pl.ds(s * RPS, RPS)])


_gat_msg = pl.kernel(
    _gat_msg_body,
    out_type=jax.ShapeDtypeStruct((NC, NP, 32), jnp.float32),
    mesh=_mesh,
    compiler_params=pltpu.CompilerParams(use_tc_tiling_on_sc=False, needs_layout_passes=False),
    scratch_types=[
        pltpu.VMEM((4, C), jnp.int32),
        pltpu.VMEM((4, C), jnp.int32),
        pltpu.VMEM((4 * C, 2), jnp.float32),
        pltpu.VMEM((2, C, 64), jnp.float32),
        pltpu.VMEM((2, C, 2), jnp.float32),
        pltpu.VMEM((2 * C,), jnp.float32),
        pltpu.VMEM((2, C, 32), jnp.float32),
        pltpu.VMEM_SHARED((NP, 32), jnp.float32),
        pltpu.SemaphoreType.DMA,
        pltpu.SemaphoreType.DMA,
        pltpu.SemaphoreType.DMA,
        pltpu.SemaphoreType.DMA,
    ],
)


def _gconv_body(h2a, h2b, srcr, dstr, out,
                idx_s, idx_d, rows, zbuf, accum, sem0, sem1):
    c = lax.axis_index("c")
    s = lax.axis_index("s")
    _zero_spmem(zbuf, accum, s, RPS, 32)
    plsc.subcore_barrier()
    nch_g = EP // C_GCV // NS
    base = s * nch_g

    def prep(slot, j, sem):
        pltpu.sync_copy(srcr.at[base + j], idx_s.at[slot])
        pltpu.sync_copy(dstr.at[base + j], idx_d.at[slot])

        @pl.when(c == 0)
        def _():
            pltpu.async_copy(h2a.at[idx_s.at[slot]], rows.at[slot], sem)

        @pl.when(c == 1)
        def _():
            pltpu.async_copy(h2b.at[idx_s.at[slot]], rows.at[slot], sem)

    def drain(slot, sem):
        pltpu.make_async_copy(h2a.at[idx_s.at[slot]], rows.at[slot],
                              sem).wait()
        pltpu.sync_copy(rows.at[slot], accum.at[idx_d.at[slot]], add=True)

    prep(0, 0, sem0)

    @pl.loop(0, nch_g // 2)
    def _(t):
        prep(1, 2 * t + 1, sem1)
        drain(0, sem0)

        @pl.when(t < nch_g // 2 - 1)
        def _():
            prep(0, 2 * t + 2, sem0)

        drain(1, sem1)

    plsc.subcore_barrier()
    pltpu.sync_copy(accum.at[pl.ds(s * RPS, RPS)],
                    out.at[c, pl.ds(s * RPS, RPS)])


_gconv = pl.kernel(
    _gconv_body,
    out_type=jax.ShapeDtypeStruct((NC, NP, 32), jnp.float32),
    mesh=_mesh,
    compiler_params=pltpu.CompilerParams(use_tc_tiling_on_sc=False, needs_layout_passes=False),
    scratch_types=[
        pltpu.VMEM((2, C_GCV), jnp.int32),
        pltpu.VMEM((2, C_GCV), jnp.int32),
        pltpu.VMEM((2, C_GCV, 32), jnp.float32),
        pltpu.VMEM((ZR, 32), jnp.float32),
        pltpu.VMEM_SHARED((NP, 32), jnp.float32),
        pltpu.SemaphoreType.DMA,
        pltpu.SemaphoreType.DMA,
    ],
)


def _pool_body(h3, bidx, sum_out, max_out, cnt_out,
               h3buf, bbuf, sacc, macc, cacc):
    c = lax.axis_index("c")
    s = lax.axis_index("s")
    w = c * NS + s
    _fill2d(sacc, BP, 64, 0.0)
    _fill2d(macc, BP, 64, float("-inf"))
    _fill2d(cacc, BP, 16, 0.0)

    it = _iota16()
    lane0 = it == 0
    ones = jnp.ones((16,), jnp.float32)

    @pl.loop(0, PN_W // PCH)
    def _(t):
        row0 = w * PN_W + t * PCH
        pltpu.sync_copy(h3.at[pl.ds(row0, PCH)], h3buf)
        pltpu.sync_copy(bidx.at[pl.ds(row0, PCH)], bbuf)

        @pl.loop(0, PCH)
        def _(i):
            vi = jnp.full((16,), i, jnp.int32)
            vb = plsc.load_gather(bbuf, [vi])
            plsc.addupdate_scatter(cacc, [vb, it], ones, mask=lane0)
            for k in range(4):
                col = k * 16 + it
                v = plsc.load_gather(h3buf, [vi, col])
                plsc.addupdate_scatter(sacc, [vb, col], v)
                cur = plsc.load_gather(macc, [vb, col])
                plsc.store_scatter(macc, [vb, col], jnp.maximum(cur, v))

    pltpu.sync_copy(sacc, sum_out.at[w])
    pltpu.sync_copy(macc, max_out.at[w])
    pltpu.sync_copy(cacc, cnt_out.at[w])


_pool = pl.kernel(
    _pool_body,
    out_type=(jax.ShapeDtypeStruct((NW, BP, 64), jnp.float32),
              jax.ShapeDtypeStruct((NW, BP, 64), jnp.float32),
              jax.ShapeDtypeStruct((NW, BP, 16), jnp.float32)),
    mesh=_mesh,
    compiler_params=pltpu.CompilerParams(use_tc_tiling_on_sc=False, needs_layout_passes=False),
    scratch_types=[
        pltpu.VMEM((PCH, 64), jnp.float32),
        pltpu.VMEM((PCH,), jnp.int32),
        pltpu.VMEM((BP, 64), jnp.float32),
        pltpu.VMEM((BP, 64), jnp.float32),
        pltpu.VMEM((BP, 16), jnp.float32),
    ],
)


def _gelu(v):
    return 0.5 * v * (1.0 + lax.erf(v * 0.7071067811865476))


BR = 1568
_GRID = NP // BR


def _tc1_body(x16, aggp, wg, bg, wgat, asrc, adst,
              y0_o, y1_o, as_o, ad_o, es_o):
    agg = aggp[0] + aggp[1]
    h = _gelu((x16[...] + agg) @ wg[...] + bg[...])
    wx = h @ wgat[...]
    wx0 = wx[:, :64]
    wx1 = wx[:, 64:]
    as0 = jnp.sum(wx0 * asrc[0, :][None, :], axis=1, keepdims=True)
    as1 = jnp.sum(wx1 * asrc[1, :][None, :], axis=1, keepdims=True)
    ad0 = jnp.sum(wx0 * adst[0, :][None, :], axis=1, keepdims=True)
    ad1 = jnp.sum(wx1 * adst[1, :][None, :], axis=1, keepdims=True)
    a_s = jnp.concatenate([as0, as1], axis=1)
    a_d = jnp.concatenate([ad0, ad1], axis=1)
    e = a_s + a_d
    es_o[...] = jnp.exp(jnp.maximum(e, 0.2 * e))
    as_o[...] = a_s
    ad_o[...] = a_d
    y0_o[...] = jnp.concatenate([wx0[:, :32], wx1[:, :32]], axis=1)
    y1_o[...] = jnp.concatenate([wx0[:, 32:], wx1[:, 32:]], axis=1)


def _tc1(x16, aggp, wg, bg, wgat, asrc, adst):
    full = lambda *shape: pl.BlockSpec(shape, lambda i: (0,) * len(shape))
    return pl.pallas_call(
        _tc1_body,
        grid=(_GRID,),
        in_specs=[
            pl.BlockSpec((BR, 16), lambda i: (i, 0)),
            pl.BlockSpec((NC, BR, 16), lambda i: (0, i, 0)),
            full(16, 64), full(64,), full(64, 128), full(2, 64), full(2, 64),
        ],
        out_specs=[
            pl.BlockSpec((BR, 64), lambda i: (i, 0)),
            pl.BlockSpec((BR, 64), lambda i: (i, 0)),
            pl.BlockSpec((BR, 2), lambda i: (i, 0)),
            pl.BlockSpec((BR, 2), lambda i: (i, 0)),
            pl.BlockSpec((BR, 2), lambda i: (i, 0)),
        ],
        out_shape=[
            jax.ShapeDtypeStruct((NP, 64), jnp.float32),
            jax.ShapeDtypeStruct((NP, 64), jnp.float32),
            jax.ShapeDtypeStruct((NP, 2), jnp.float32),
            jax.ShapeDtypeStruct((NP, 2), jnp.float32),
            jax.ShapeDtypeStruct((NP, 2), jnp.float32),
        ],
    )(x16, aggp, wg, bg, wgat, asrc, adst)


def _tcden_body(denp, es, den_o):
    den_o[...] = denp[0] + denp[1] + es[...]


def _tcden(denp, es):
    return pl.pallas_call(
        _tcden_body,
        grid=(_GRID,),
        in_specs=[
            pl.BlockSpec((NC, BR, 2), lambda i: (0, i, 0)),
            pl.BlockSpec((BR, 2), lambda i: (i, 0)),
        ],
        out_specs=pl.BlockSpec((BR, 2), lambda i: (i, 0)),
        out_shape=jax.ShapeDtypeStruct((NP, 2), jnp.float32),
    )(denp, es)


def _tc2_body(nump, den, es, y0, y1, bgat, h2a_o, h2b_o):
    num = jnp.concatenate([nump[0], nump[1]], axis=1)
    wx_h0 = jnp.concatenate([y0[:, :32], y1[:, :32]], axis=1)
    wx_h1 = jnp.concatenate([y0[:, 32:], y1[:, 32:]], axis=1)
    s0 = es[:, 0:1] / den[:, 0:1]
    s1 = es[:, 1:2] / den[:, 1:2]
    h2 = _gelu(0.5 * (num + s0 * wx_h0 + s1 * wx_h1) + bgat[...])
    h2a_o[...] = h2[:, :32]
    h2b_o[...] = h2[:, 32:]


def _tc2(nump, den, es, y0, y1, bgat):
    return pl.pallas_call(
        _tc2_body,
        grid=(_GRID,),
        in_specs=[
            pl.BlockSpec((NC, BR, 32), lambda i: (0, i, 0)),
            pl.BlockSpec((BR, 2), lambda i: (i, 0)),
            pl.BlockSpec((BR, 2), lambda i: (i, 0)),
            pl.BlockSpec((BR, 64), lambda i: (i, 0)),
            pl.BlockSpec((BR, 64), lambda i: (i, 0)),
            pl.BlockSpec((64,), lambda i: (0,)),
        ],
        out_specs=[
            pl.BlockSpec((BR, 32), lambda i: (i, 0)),
            pl.BlockSpec((BR, 32), lambda i: (i, 0)),
        ],
        out_shape=[
            jax.ShapeDtypeStruct((NP, 32), jnp.float32),
            jax.ShapeDtypeStruct((NP, 32), jnp.float32),
        ],
    )(nump, den, es, y0, y1, bgat)


def _tc3_body(aggp, h2a, h2b, wrel, brel, wroot, h3_o):
    agg3 = jnp.concatenate([aggp[0], aggp[1]], axis=1)
    h2 = jnp.concatenate([h2a[...], h2b[...]], axis=1)
    h3_o[...] = _gelu(agg3 @ wrel[...] + brel[...] + h2 @ wroot[...])


def _tc3(aggp, h2a, h2b, wrel, brel, wroot):
    full = lambda *shape: pl.BlockSpec(shape, lambda i: (0,) * len(shape))
    return pl.pallas_call(
        _tc3_body,
        grid=(_GRID,),
        in_specs=[
            pl.BlockSpec((NC, BR, 32), lambda i: (0, i, 0)),
            pl.BlockSpec((BR, 32), lambda i: (i, 0)),
            pl.BlockSpec((BR, 32), lambda i: (i, 0)),
            full(64, 64), full(64,), full(64, 64),
        ],
        out_specs=pl.BlockSpec((BR, 64), lambda i: (i, 0)),
        out_shape=jax.ShapeDtypeStruct((NP, 64), jnp.float32),
    )(aggp, h2a, h2b, wrel, brel, wroot)


def _tc4_body(sum_p, max_p, cnt_p, desc, wsh, bsh, wtg1, btg1, wtg2, btg2,
              wtm1, btm1, wtm2, btm2, out_o, shared_o):
    sums = jnp.sum(sum_p[:, :B, :], axis=0)
    maxs = jnp.max(max_p[:, :B, :], axis=0)
    cnt = jnp.sum(cnt_p[:, :B, 0], axis=0)[:, None]
    mean_p = sums / jnp.maximum(cnt, 1.0)
    maxv = jnp.where(cnt > 0, maxs, 0.0)
    combined = jnp.concatenate([maxv, mean_p, desc[...]], axis=1)
    shared = _gelu(combined @ wsh[...] + bsh[...])
    tg = _gelu(shared @ wtg1[...] + btg1[...]) @ wtg2[...] + btg2[...]
    tm = _gelu(shared @ wtm1[...] + btm1[...]) @ wtm2[...] + btm2[...]
    out_o[...] = jnp.concatenate([tg, tm], axis=1)
    shared_o[...] = shared


def _tc4(sum_p, max_p, cnt_p, desc, wsh, bsh, wtg1, btg1, wtg2, btg2,
         wtm1, btm1, wtm2, btm2):
    return pl.pallas_call(
        _tc4_body,
        out_shape=[
            jax.ShapeDtypeStruct((B, 2), jnp.float32),
            jax.ShapeDtypeStruct((B, 128), jnp.float32),
        ],
    )(sum_p, max_p, cnt_p, desc, wsh, bsh, wtg1, btg1, wtg2, btg2,
      wtm1, btm1, wtm2, btm2)


_seg_sum_gin = _make_seg_sum(16, split_by_worker=True, CK=C_GIN)


def kernel(x, edge_index, batch_index, descriptors, W_gin, b_gin, W_gat,
           att_src, att_dst, b_gat, W_rel, b_rel, W_root, W_sh, b_sh,
           W_tg1, b_tg1, W_tg2, b_tg2, W_tm1, b_tm1, W_tm2, b_tm2):
    src = edge_index[0]
    dst = edge_index[1]
    pad_e = jnp.full((EP - E,), N, jnp.int32)
    srcf = jnp.concatenate([src, pad_e])
    dstf = jnp.concatenate([dst, pad_e])
    srcr = srcf.reshape(NCHUNKS, C)
    dstr = dstf.reshape(NCHUNKS, C)
    src_g = srcf.reshape(EP // C_GIN, C_GIN)
    dst_g = dstf.reshape(EP // C_GIN, C_GIN)
    src_a = srcf.reshape(EP // C_SCA, C_SCA)
    dst_a = dstf.reshape(EP // C_SCA, C_SCA)
    src_c = srcf.reshape(EP // C_GCV, C_GCV)
    dst_c = dstf.reshape(EP // C_GCV, C_GCV)
    x16 = jnp.zeros((NP, 16), jnp.float32).at[:N, :9].set(x)
    bpad = jnp.concatenate(
        [batch_index, jnp.full((NP - N,), B, jnp.int32)])
    wg16 = jnp.zeros((16, 64), jnp.float32).at[:9].set(W_gin)

    aggp = _seg_sum_gin(x16, src_g, dst_g)
    y0, y1, a_s, a_d, es = _tc1(x16, aggp, wg16, b_gin, W_gat,
                                att_src, att_dst)
    ee, denp = _gat_edge(a_s, a_d, src_a, dst_a)
    den = _tcden(denp, es)
    z32 = jnp.zeros((NP, 32), jnp.float32)
    nump = _gat_msg(y0, y1, ee, den, z32, srcr, dstr)
    h2a, h2b = _tc2(nump, den, es, y0, y1, b_gat)
    agg3p = _gconv(h2a, h2b, src_c, dst_c)
    h3 = _tc3(agg3p, h2a, h2b, W_rel, b_rel, W_root)
    sum_p, max_p, cnt_p = _pool(h3, bpad)
    out, shared = _tc4(sum_p, max_p, cnt_p, descriptors, W_sh, b_sh,
                       W_tg1, b_tg1, W_tg2, b_tg2, W_tm1, b_tm1,
                       W_tm2, b_tm2)
    return out, shared
